# Initial kernel scaffold; baseline (speedup 1.0000x reference)
import jax
import jax.numpy as jnp
from jax.experimental import pallas as pl


def kernel(user_feats, graph_node_features, graph_edge_index, merged_tree_feature, merged_tree_edge_index, indices, emb_tree, emb_graph, h0_tree, h0_graph, ue_W1, ue_b1, ue_W2, ue_b2, gt_Wih0, gt_Whh0, gt_bih0, gt_bhh0, gt_Wih1, gt_Whh1, gt_bih1, gt_bhh1, gg_Wih0, gg_Whh0, gg_bih0, gg_bhh0, gg_Wih1, gg_Whh1, gg_bih1, gg_bhh1, tc1_W, tc1_b, tc2_W, tc2_b, gc1_W, gc1_b, gc2_W, gc2_b, fc_W, fc_b):
    raise NotImplementedError("write your pallas kernel here")


# speedup vs baseline: 5.0803x; 5.0803x over previous
import functools

import jax
import jax.numpy as jnp
from jax import lax
from jax.experimental import pallas as pl
from jax.experimental.pallas import tpu as pltpu
from jax.experimental.pallas import tpu_sc as plsc

F32 = jnp.float32

_N_USERS = 2048
_N_GT = 4096
_N_TREE = 2048
_VOCAB = 30000
_D = 100
_H = 100
_SEQ = 20
_E_GRAPH = 65536
_E_TREE = 2048
_BATCH = 32
_N_GRAPH = _N_GT + _N_USERS

_DP = 128
_GP = 128
_G3 = 3 * _GP

_NC = 2
_NS = 16
_NW = _NC * _NS


def _sc_mesh():
    return plsc.VectorSubcoreMesh(core_axis_name="c", subcore_axis_name="s",
                                  num_cores=_NC, num_subcores=_NS)


def _sc_gather(table, idx, B, Dp):
    bpw = B // _NW
    K = min(128, bpw)
    nch = bpw // K

    def body(table_hbm, idx_hbm, out_hbm, idx_v, rows_v, sem):
        c = lax.axis_index("c")
        s = lax.axis_index("s")
        base = (s * _NC + c) * bpw

        def step(j, carry):
            off = base + j * K
            pltpu.sync_copy(idx_hbm.at[pl.ds(off, K)], idx_v)
            pltpu.async_copy(table_hbm.at[idx_v], rows_v, sem).wait()
            pltpu.sync_copy(rows_v, out_hbm.at[pl.ds(off, K)])
            return carry

        lax.fori_loop(0, nch, step, 0)

    k = pl.kernel(
        body,
        out_type=jax.ShapeDtypeStruct((B, Dp), F32),
        mesh=_sc_mesh(),
        scratch_types=[pltpu.VMEM((K,), jnp.int32),
                       pltpu.VMEM((K, Dp), F32),
                       pltpu.SemaphoreType.DMA])
    return k(table, idx)


def _sc_agg(y, src, dst, N, F, E):
    epw = E // _NW
    K = min(128, epw)
    nch = epw // K
    rpt = N // _NS
    zeros = jnp.zeros((N, F), F32)

    def body(y_hbm, src_hbm, dst_hbm, z_hbm, out_hbm,
             si_v, di_v, rows_v, acc_sh, sem):
        c = lax.axis_index("c")
        s = lax.axis_index("s")
        base = (s * _NC + c) * epw
        zslc = pl.ds(s * rpt, rpt)
        pltpu.sync_copy(z_hbm.at[zslc], acc_sh.at[zslc])
        plsc.subcore_barrier()

        def step(j, carry):
            off = base + j * K
            pltpu.sync_copy(src_hbm.at[pl.ds(off, K)], si_v)
            pltpu.sync_copy(dst_hbm.at[pl.ds(off, K)], di_v)
            pltpu.async_copy(y_hbm.at[si_v], rows_v, sem).wait()
            pltpu.sync_copy(rows_v, acc_sh.at[di_v], add=True)
            return carry

        lax.fori_loop(0, nch, step, 0)
        plsc.subcore_barrier()
        pltpu.sync_copy(acc_sh.at[zslc], out_hbm.at[c, zslc])

    k = pl.kernel(
        body,
        out_type=jax.ShapeDtypeStruct((_NC, N, F), F32),
        mesh=_sc_mesh(),
        scratch_types=[pltpu.VMEM((K,), jnp.int32),
                       pltpu.VMEM((K,), jnp.int32),
                       pltpu.VMEM((K, F), F32),
                       pltpu.VMEM_SHARED((N, F), F32),
                       pltpu.SemaphoreType.DMA])
    return k(y, src, dst, zeros)


def _sc_deg(dst, N, E):
    epw = E // _NW
    K = min(128, epw)
    nch = epw // K
    rpt = N // _NS
    ones = jnp.ones((K, _DP), F32)
    zeros = jnp.zeros((N, _DP), F32)

    def body(ones_hbm, z_hbm, dst_hbm, out_hbm, di_v, ones_v, acc_sh):
        c = lax.axis_index("c")
        s = lax.axis_index("s")
        base = (s * _NC + c) * epw
        pltpu.sync_copy(ones_hbm, ones_v)
        zslc = pl.ds(s * rpt, rpt)
        pltpu.sync_copy(z_hbm.at[zslc], acc_sh.at[zslc])
        plsc.subcore_barrier()

        def step(j, carry):
            off = base + j * K
            pltpu.sync_copy(dst_hbm.at[pl.ds(off, K)], di_v)
            pltpu.sync_copy(ones_v, acc_sh.at[di_v], add=True)
            return carry

        lax.fori_loop(0, nch, step, 0)
        plsc.subcore_barrier()
        pltpu.sync_copy(acc_sh.at[zslc], out_hbm.at[c, zslc])

    k = pl.kernel(
        body,
        out_type=jax.ShapeDtypeStruct((_NC, N, _DP), F32),
        mesh=_sc_mesh(),
        scratch_types=[pltpu.VMEM((K,), jnp.int32),
                       pltpu.VMEM((K, _DP), F32),
                       pltpu.VMEM_SHARED((N, _DP), F32)])
    return k(ones, zeros, dst)


def _mm_bias(x, w, b, bm):
    B, Kd = x.shape
    N = w.shape[1]

    def kfn(x_ref, w_ref, b_ref, o_ref):
        o_ref[...] = jnp.dot(x_ref[...], w_ref[...],
                             preferred_element_type=F32) + b_ref[...]

    return pl.pallas_call(
        kfn,
        grid=(B // bm,),
        in_specs=[pl.BlockSpec((bm, Kd), lambda i: (i, 0)),
                  pl.BlockSpec((Kd, N), lambda i: (0, 0)),
                  pl.BlockSpec((1, N), lambda i: (0, 0))],
        out_specs=pl.BlockSpec((bm, N), lambda i: (i, 0)),
        out_shape=jax.ShapeDtypeStruct((B, N), F32))(x, w, b[None])


def _gru2_tc(gi, h01, h02, whh0, wih1, whh1, bhh0, bih1, bhh1, nb):
    S, N, G3 = gi.shape

    def kfn(gi_ref, h01_ref, h02_ref, w0_ref, w1_ref, w2_ref,
            b0_ref, b1_ref, b2_ref, o_ref):
        h1 = h01_ref[...]
        h2 = h02_ref[...]
        w0 = w0_ref[...]
        w1 = w1_ref[...]
        w2 = w2_ref[...]
        b0 = b0_ref[...]
        b1 = b1_ref[...]
        b2 = b2_ref[...]

        def gate(gi_t, gh_t, h):
            r = jax.nn.sigmoid(gi_t[:, 0:_GP] + gh_t[:, 0:_GP])
            z = jax.nn.sigmoid(gi_t[:, _GP:2 * _GP] + gh_t[:, _GP:2 * _GP])
            n = jnp.tanh(gi_t[:, 2 * _GP:] + r * gh_t[:, 2 * _GP:])
            return (1.0 - z) * n + z * h

        for t in range(S):
            gh1 = jnp.dot(h1, w0, preferred_element_type=F32) + b0
            h1 = gate(gi_ref[t], gh1, h1)
            gi2 = jnp.dot(h1, w1, preferred_element_type=F32) + b1
            gh2 = jnp.dot(h2, w2, preferred_element_type=F32) + b2
            h2 = gate(gi2, gh2, h2)
        o_ref[...] = h2

    wspec = pl.BlockSpec((_GP, G3), lambda i: (0, 0))
    bspec = pl.BlockSpec((1, G3), lambda i: (0, 0))
    hspec = pl.BlockSpec((nb, _GP), lambda i: (i, 0))
    return pl.pallas_call(
        kfn,
        grid=(N // nb,),
        in_specs=[pl.BlockSpec((S, nb, G3), lambda i: (0, i, 0)),
                  hspec, hspec, wspec, wspec, wspec, bspec, bspec, bspec],
        out_specs=hspec,
        out_shape=jax.ShapeDtypeStruct((N, _GP), F32))(
            gi, h01, h02, whh0, wih1, whh1,
            bhh0[None], bih1[None], bhh1[None])


def _ue_tc(uf, w1, b1, w2, b2):
    def kfn(x_ref, w1_ref, b1_ref, w2_ref, b2_ref, o_ref):
        h = jnp.maximum(jnp.dot(x_ref[...], w1_ref[...],
                                preferred_element_type=F32) + b1_ref[...], 0.0)
        o_ref[...] = jnp.dot(h, w2_ref[...],
                             preferred_element_type=F32) + b2_ref[...]

    return pl.pallas_call(
        kfn,
        out_shape=jax.ShapeDtypeStruct((uf.shape[0], w2.shape[1]), F32))(
            uf, w1, b1[None], w2, b2[None])


def _dinv_of(d):
    return lax.rsqrt(d[0, :, 0:1] + d[1, :, 0:1] + 1.0)


def _elu(x):
    return jnp.where(x > 0, x, jnp.exp(jnp.minimum(x, 0.0)) - 1.0)


def _gcn_pre(x, w, degp, bm):
    B, Kd = x.shape
    Fo = w.shape[1]

    def kfn(x_ref, w_ref, d_ref, o_ref):
        o_ref[...] = _dinv_of(d_ref[...]) * jnp.dot(
            x_ref[...], w_ref[...], preferred_element_type=F32)

    return pl.pallas_call(
        kfn,
        grid=(B // bm,),
        in_specs=[pl.BlockSpec((bm, Kd), lambda i: (i, 0)),
                  pl.BlockSpec((Kd, Fo), lambda i: (0, 0)),
                  pl.BlockSpec((2, bm, _DP), lambda i: (0, i, 0))],
        out_specs=pl.BlockSpec((bm, Fo), lambda i: (i, 0)),
        out_shape=jax.ShapeDtypeStruct((B, Fo), F32))(x, w, degp)


def _gcn_mid_graph(p, y1, degp, b1, w2, bm):
    _, B, F1 = p.shape
    F2 = w2.shape[1]

    def kfn(p_ref, y_ref, d_ref, b_ref, w_ref, o_ref):
        dinv = _dinv_of(d_ref[...])
        xg = _elu(dinv * (p_ref[0] + p_ref[1] + y_ref[...]) + b_ref[...])
        o_ref[...] = dinv * jnp.dot(xg, w_ref[...], preferred_element_type=F32)

    return pl.pallas_call(
        kfn,
        grid=(B // bm,),
        in_specs=[pl.BlockSpec((2, bm, F1), lambda i: (0, i, 0)),
                  pl.BlockSpec((bm, F1), lambda i: (i, 0)),
                  pl.BlockSpec((2, bm, _DP), lambda i: (0, i, 0)),
                  pl.BlockSpec((1, F1), lambda i: (0, 0)),
                  pl.BlockSpec((F1, F2), lambda i: (0, 0))],
        out_specs=pl.BlockSpec((bm, F2), lambda i: (i, 0)),
        out_shape=jax.ShapeDtypeStruct((B, F2), F32))(p, y1, degp, b1, w2)


def _graph_head(p, y2, degp, b2, fcw, fcb):
    _, _, F2 = p.shape
    C = fcw.shape[1]

    def kfn(p_ref, y_ref, d_ref, b_ref, w_ref, fb_ref, o_ref):
        dinv = _dinv_of(d_ref[...])
        xg = _elu(dinv * (p_ref[0] + p_ref[1] + y_ref[...]) + b_ref[...])
        o_ref[...] = jnp.dot(xg, w_ref[...],
                             preferred_element_type=F32) + fb_ref[...]

    return pl.pallas_call(
        kfn,
        grid=(1,),
        in_specs=[pl.BlockSpec((2, _BATCH, F2), lambda i: (0, 0, 0)),
                  pl.BlockSpec((_BATCH, F2), lambda i: (0, 0)),
                  pl.BlockSpec((2, _BATCH, _DP), lambda i: (0, 0, 0)),
                  pl.BlockSpec((1, F2), lambda i: (0, 0)),
                  pl.BlockSpec((F2, C), lambda i: (0, 0)),
                  pl.BlockSpec((1, C), lambda i: (0, 0))],
        out_specs=pl.BlockSpec((_BATCH, C), lambda i: (0, 0)),
        out_shape=jax.ShapeDtypeStruct((_BATCH, C), F32))(
            p, y2, degp, b2, fcw, fcb)


def _tree_mid(p, y1, degp, b1, x1head, idxcol, wa, wb):
    N = y1.shape[0]

    def kfn(p_ref, y_ref, d_ref, b_ref, xh_ref, idx_ref, wa_ref, wb_ref, o_ref):
        dinv = _dinv_of(d_ref[...])
        xca = _elu(dinv * (p_ref[0] + p_ref[1] + y_ref[...]) + b_ref[...])
        cols = lax.broadcasted_iota(jnp.int32, (N, _BATCH), 1)
        oh = (cols == idx_ref[...]).astype(F32)
        xcb = _elu(jnp.dot(oh, xh_ref[...], preferred_element_type=F32))
        o_ref[...] = dinv * (
            jnp.dot(xca, wa_ref[...], preferred_element_type=F32)
            + jnp.dot(xcb, wb_ref[...], preferred_element_type=F32))

    return pl.pallas_call(
        kfn,
        out_shape=jax.ShapeDtypeStruct((N, _DP), F32))(
            p, y1, degp, b1, x1head, idxcol, wa, wb)


def _tree_post(p, y2, degp, b2, idxrow):
    N = y2.shape[0]

    def kfn(p_ref, y_ref, d_ref, b_ref, idx_ref, o_ref):
        dinv = _dinv_of(d_ref[...])
        xc2 = _elu(dinv * (p_ref[0] + p_ref[1] + y_ref[...]) + b_ref[...])
        rows = lax.broadcasted_iota(jnp.int32, (_BATCH, N), 0)
        oht = (rows == idx_ref[...]).astype(F32)
        seg = jnp.dot(oht, xc2, preferred_element_type=F32)
        cnt = jnp.sum(oht, axis=1, keepdims=True)
        o_ref[...] = seg / cnt

    return pl.pallas_call(
        kfn,
        out_shape=jax.ShapeDtypeStruct((_BATCH, _DP), F32))(
            p, y2, degp, b2, idxrow)


def _pad2(a, r, c):
    return jnp.pad(a, ((0, r - a.shape[0]), (0, c - a.shape[1])))


def _gates_T(W, kpad):
    wt = W.T.reshape(W.shape[1], 3, _H)
    wt = jnp.pad(wt, ((0, kpad - W.shape[1]), (0, 0), (0, _GP - _H)))
    return wt.reshape(kpad, _G3)


def _gates_b(b):
    return jnp.pad(b.reshape(3, _H), ((0, 0), (0, _GP - _H))).reshape(_G3)


def kernel(user_feats, graph_node_features, graph_edge_index,
           merged_tree_feature, merged_tree_edge_index, indices,
           emb_tree, emb_graph, h0_tree, h0_graph,
           ue_W1, ue_b1, ue_W2, ue_b2,
           gt_Wih0, gt_Whh0, gt_bih0, gt_bhh0,
           gt_Wih1, gt_Whh1, gt_bih1, gt_bhh1,
           gg_Wih0, gg_Whh0, gg_bih0, gg_bhh0,
           gg_Wih1, gg_Whh1, gg_bih1, gg_bhh1,
           tc1_W, tc1_b, tc2_W, tc2_b,
           gc1_W, gc1_b, gc2_W, gc2_b,
           fc_W, fc_b):
    i32 = jnp.int32
    tree_tok = merged_tree_feature.astype(i32).T.reshape(-1)
    graph_tok = graph_node_features.astype(i32).T.reshape(-1)
    embt_p = _pad2(emb_tree, _VOCAB, _DP)
    embg_p = _pad2(emb_graph, _VOCAB, _DP)
    g_src = graph_edge_index[0].astype(i32)
    g_dst = graph_edge_index[1].astype(i32)
    t_src = merged_tree_edge_index[1].astype(i32)
    t_dst = merged_tree_edge_index[0].astype(i32)
    idx_i = indices.astype(i32)
    h0t = jnp.pad(h0_tree, ((0, 0), (0, 0), (0, _GP - _H)))
    h0g = jnp.pad(h0_graph, ((0, 0), (0, 0), (0, _GP - _H)))

    wih0_t = _gates_T(gt_Wih0, _DP)
    whh0_t = _gates_T(gt_Whh0, _GP)
    wih1_t = _gates_T(gt_Wih1, _GP)
    whh1_t = _gates_T(gt_Whh1, _GP)
    wih0_g = _gates_T(gg_Wih0, _DP)
    whh0_g = _gates_T(gg_Whh0, _GP)
    wih1_g = _gates_T(gg_Wih1, _GP)
    whh1_g = _gates_T(gg_Whh1, _GP)
    bih0_t = _gates_b(gt_bih0)
    bhh0_t = _gates_b(gt_bhh0)
    bih1_t = _gates_b(gt_bih1)
    bhh1_t = _gates_b(gt_bhh1)
    bih0_g = _gates_b(gg_bih0)
    bhh0_g = _gates_b(gg_bhh0)
    bih1_g = _gates_b(gg_bih1)
    bhh1_g = _gates_b(gg_bhh1)

    tc1_Wp = _pad2(tc1_W, _GP, _DP)
    tc2_Wa = _pad2(tc2_W[:_H], _DP, _DP)
    tc2_Wb = _pad2(tc2_W[_H:], _GP, _DP)
    tc1_bp = _pad2(tc1_b[None], 1, _DP)
    tc2_bp = _pad2(tc2_b[None], 1, _DP)
    gc2_Wp = _pad2(gc2_W, _DP, _DP)
    gc2_bp = _pad2(gc2_b[None], 1, _DP)
    fc_Wp = _pad2(fc_W, _DP, fc_W.shape[1])

    degp_g = _sc_deg(g_dst, _N_GRAPH, _E_GRAPH)
    degp_t = _sc_deg(t_dst, _N_TREE, _E_TREE)
    xt = _sc_gather(embt_p, tree_tok, _SEQ * _N_TREE, _DP)
    xg = _sc_gather(embg_p, graph_tok, _SEQ * _N_GT, _DP)

    git = _mm_bias(xt, wih0_t, bih0_t, 2048).reshape(_SEQ, _N_TREE, _G3)
    gig = _mm_bias(xg, wih0_g, bih0_g, 2048).reshape(_SEQ, _N_GT, _G3)
    x1 = _gru2_tc(git, h0t[0], h0t[1], whh0_t, wih1_t, whh1_t,
                  bhh0_t, bih1_t, bhh1_t, 256)
    hng = _gru2_tc(gig, h0g[0], h0g[1], whh0_g, wih1_g, whh1_g,
                   bhh0_g, bih1_g, bhh1_g, 256)

    y1t = _gcn_pre(x1, tc1_Wp, degp_t, 2048)
    tp1 = _sc_agg(y1t, t_src, t_dst, _N_TREE, _DP, _E_TREE)
    y2t = _tree_mid(tp1, y1t, degp_t, tc1_bp, x1[:_BATCH],
                    idx_i[:, None], tc2_Wa, tc2_Wb)
    tp2 = _sc_agg(y2t, t_src, t_dst, _N_TREE, _DP, _E_TREE)
    temb = _tree_post(tp2, y2t, degp_t, tc2_bp, idx_i[None, :])

    ue = _ue_tc(user_feats, ue_W1, ue_b1, ue_W2, ue_b2)
    x_input = jnp.concatenate(
        [temb[:, :_H], ue, hng[_BATCH:, :_H]], axis=0)
    y1g = _gcn_pre(x_input, _pad2(gc1_W, gc1_W.shape[0], _DP), degp_g, 1024)
    gp1 = _sc_agg(y1g, g_src, g_dst, _N_GRAPH, _DP, _E_GRAPH)
    y2g = _gcn_mid_graph(gp1, y1g, degp_g, _pad2(gc1_b[None], 1, _DP), gc2_Wp, 1024)
    gp2 = _sc_agg(y2g, g_src, g_dst, _N_GRAPH, _DP, _E_GRAPH)
    out = _graph_head(gp2, y2g, degp_g, gc2_bp, fc_Wp, fc_b[None])
    return out


# speedup vs baseline: 6.9422x; 1.3665x over previous
import functools

import jax
import jax.numpy as jnp
from jax import lax
from jax.experimental import pallas as pl
from jax.experimental.pallas import tpu as pltpu
from jax.experimental.pallas import tpu_sc as plsc

F32 = jnp.float32

_N_USERS = 2048
_N_GT = 4096
_N_TREE = 2048
_VOCAB = 30000
_D = 100
_H = 100
_SEQ = 20
_E_GRAPH = 65536
_E_TREE = 2048
_BATCH = 32
_N_GRAPH = _N_GT + _N_USERS

_DP = 128
_GP = 128
_G3 = 3 * _GP

_NC = 2
_NS = 16
_NW = _NC * _NS


def _sc_mesh():
    return plsc.VectorSubcoreMesh(core_axis_name="c", subcore_axis_name="s",
                                  num_cores=_NC, num_subcores=_NS)


def _sc_gather(table, idx, B, Dp):
    bpw = B // _NW
    K = min(128, bpw)
    nch = bpw // K

    def body(table_hbm, idx_hbm, out_hbm, idx_v, rows_v, sem):
        c = lax.axis_index("c")
        s = lax.axis_index("s")
        base = (s * _NC + c) * bpw

        def step(j, carry):
            off = base + j * K
            pltpu.sync_copy(idx_hbm.at[pl.ds(off, K)], idx_v)
            pltpu.async_copy(table_hbm.at[idx_v], rows_v, sem).wait()
            pltpu.sync_copy(rows_v, out_hbm.at[pl.ds(off, K)])
            return carry

        lax.fori_loop(0, nch, step, 0)

    k = pl.kernel(
        body,
        out_type=jax.ShapeDtypeStruct((B, Dp), F32),
        mesh=_sc_mesh(),
        scratch_types=[pltpu.VMEM((K,), jnp.int32),
                       pltpu.VMEM((K, Dp), F32),
                       pltpu.SemaphoreType.DMA])
    return k(table, idx)


def _sc_agg(y, src, dst, N, F, E):
    epw = E // _NW
    K = min(128, epw)
    nch = epw // K
    rpt = N // _NS
    zeros = jnp.zeros((N, F), F32)

    def body(y_hbm, src_hbm, dst_hbm, z_hbm, out_hbm,
             si0, si1, di_v, rows0, rows1, acc_sh, sem0, sem1):
        c = lax.axis_index("c")
        s = lax.axis_index("s")
        base = (s * _NC + c) * epw
        zslc = pl.ds(s * rpt, rpt)
        pltpu.sync_copy(z_hbm.at[zslc], acc_sh.at[zslc])
        plsc.subcore_barrier()

        def start(j, si, rows, sem):
            off = base + j * K
            pltpu.sync_copy(src_hbm.at[pl.ds(off, K)], si)
            pltpu.async_copy(y_hbm.at[si], rows, sem)

        def finish(j, rows):
            off = base + j * K
            pltpu.sync_copy(dst_hbm.at[pl.ds(off, K)], di_v)
            pltpu.sync_copy(rows, acc_sh.at[di_v], add=True)

        if nch == 1:
            start(0, si0, rows0, sem0)
            pltpu.make_async_copy(y_hbm.at[si0], rows0, sem0).wait()
            finish(0, rows0)
        else:
            start(0, si0, rows0, sem0)

            def step2(k2, carry):
                j0 = 2 * k2
                pltpu.make_async_copy(y_hbm.at[si0], rows0, sem0).wait()
                start(jnp.minimum(j0 + 1, nch - 1), si1, rows1, sem1)
                finish(j0, rows0)
                pltpu.make_async_copy(y_hbm.at[si1], rows1, sem1).wait()
                start(jnp.minimum(j0 + 2, nch - 1), si0, rows0, sem0)
                finish(j0 + 1, rows1)
                return carry

            lax.fori_loop(0, nch // 2, step2, 0)
            pltpu.make_async_copy(y_hbm.at[si0], rows0, sem0).wait()
        plsc.subcore_barrier()
        pltpu.sync_copy(acc_sh.at[zslc], out_hbm.at[c, zslc])

    k = pl.kernel(
        body,
        out_type=jax.ShapeDtypeStruct((_NC, N, F), F32),
        mesh=_sc_mesh(),
        scratch_types=[pltpu.VMEM((K,), jnp.int32),
                       pltpu.VMEM((K,), jnp.int32),
                       pltpu.VMEM((K,), jnp.int32),
                       pltpu.VMEM((K, F), F32),
                       pltpu.VMEM((K, F), F32),
                       pltpu.VMEM_SHARED((N, F), F32),
                       pltpu.SemaphoreType.DMA,
                       pltpu.SemaphoreType.DMA])
    return k(y, src, dst, zeros)


def _sc_deg(dst, N, E):
    epw = E // _NW
    K = min(128, epw)
    nch = epw // K
    rpt = N // _NS
    ones = jnp.ones((K, _DP), F32)
    zeros = jnp.zeros((N, _DP), F32)

    def body(ones_hbm, z_hbm, dst_hbm, out_hbm, di_v, ones_v, acc_sh):
        c = lax.axis_index("c")
        s = lax.axis_index("s")
        base = (s * _NC + c) * epw
        pltpu.sync_copy(ones_hbm, ones_v)
        zslc = pl.ds(s * rpt, rpt)
        pltpu.sync_copy(z_hbm.at[zslc], acc_sh.at[zslc])
        plsc.subcore_barrier()

        def step(j, carry):
            off = base + j * K
            pltpu.sync_copy(dst_hbm.at[pl.ds(off, K)], di_v)
            pltpu.sync_copy(ones_v, acc_sh.at[di_v], add=True)
            return carry

        lax.fori_loop(0, nch, step, 0)
        plsc.subcore_barrier()
        pltpu.sync_copy(acc_sh.at[zslc], out_hbm.at[c, zslc])

    k = pl.kernel(
        body,
        out_type=jax.ShapeDtypeStruct((_NC, N, _DP), F32),
        mesh=_sc_mesh(),
        scratch_types=[pltpu.VMEM((K,), jnp.int32),
                       pltpu.VMEM((K, _DP), F32),
                       pltpu.VMEM_SHARED((N, _DP), F32)])
    return k(ones, zeros, dst)


def _gru2_tc(x, h01, h02, wih0, whh0, wih1, whh1,
             bih0, bhh0, bih1, bhh1, nb):
    S, N, Dx = x.shape

    def kfn(x_ref, h01_ref, h02_ref, wi0_ref, w0_ref, w1_ref, w2_ref,
            bi0_ref, b0_ref, b1_ref, b2_ref, o_ref):
        h1 = h01_ref[...]
        h2 = h02_ref[...]
        wi0 = wi0_ref[...]
        w0 = w0_ref[...]
        w1 = w1_ref[...]
        w2 = w2_ref[...]
        bi0 = bi0_ref[...]
        b0 = b0_ref[...]
        b1 = b1_ref[...]
        b2 = b2_ref[...]

        def gate(gi_t, gh_t, h):
            r = jax.nn.sigmoid(gi_t[:, 0:_GP] + gh_t[:, 0:_GP])
            z = jax.nn.sigmoid(gi_t[:, _GP:2 * _GP] + gh_t[:, _GP:2 * _GP])
            n = jnp.tanh(gi_t[:, 2 * _GP:] + r * gh_t[:, 2 * _GP:])
            return (1.0 - z) * n + z * h

        for t in range(S):
            gi1 = jnp.dot(x_ref[t], wi0, preferred_element_type=F32) + bi0
            gh1 = jnp.dot(h1, w0, preferred_element_type=F32) + b0
            h1 = gate(gi1, gh1, h1)
            gi2 = jnp.dot(h1, w1, preferred_element_type=F32) + b1
            gh2 = jnp.dot(h2, w2, preferred_element_type=F32) + b2
            h2 = gate(gi2, gh2, h2)
        o_ref[...] = h2

    wispec = pl.BlockSpec((Dx, _G3), lambda i: (0, 0))
    wspec = pl.BlockSpec((_GP, _G3), lambda i: (0, 0))
    bspec = pl.BlockSpec((1, _G3), lambda i: (0, 0))
    hspec = pl.BlockSpec((nb, _GP), lambda i: (i, 0))
    return pl.pallas_call(
        kfn,
        grid=(N // nb,),
        in_specs=[pl.BlockSpec((S, nb, Dx), lambda i: (0, i, 0)),
                  hspec, hspec, wispec, wspec, wspec, wspec,
                  bspec, bspec, bspec, bspec],
        out_specs=hspec,
        out_shape=jax.ShapeDtypeStruct((N, _GP), F32))(
            x, h01, h02, wih0, whh0, wih1, whh1,
            bih0[None], bhh0[None], bih1[None], bhh1[None])


def _ue_tc(uf, w1, b1, w2, b2):
    def kfn(x_ref, w1_ref, b1_ref, w2_ref, b2_ref, o_ref):
        h = jnp.maximum(jnp.dot(x_ref[...], w1_ref[...],
                                preferred_element_type=F32) + b1_ref[...], 0.0)
        o_ref[...] = jnp.dot(h, w2_ref[...],
                             preferred_element_type=F32) + b2_ref[...]

    return pl.pallas_call(
        kfn,
        out_shape=jax.ShapeDtypeStruct((uf.shape[0], w2.shape[1]), F32))(
            uf, w1, b1[None], w2, b2[None])


def _dinv_of(d):
    return lax.rsqrt(d[0, :, 0:1] + d[1, :, 0:1] + 1.0)


def _elu(x):
    return jnp.where(x > 0, x, jnp.exp(jnp.minimum(x, 0.0)) - 1.0)


def _gcn_pre(x, w, degp, bm):
    B, Kd = x.shape
    Fo = w.shape[1]

    def kfn(x_ref, w_ref, d_ref, o_ref):
        o_ref[...] = _dinv_of(d_ref[...]) * jnp.dot(
            x_ref[...], w_ref[...], preferred_element_type=F32)

    return pl.pallas_call(
        kfn,
        grid=(B // bm,),
        in_specs=[pl.BlockSpec((bm, Kd), lambda i: (i, 0)),
                  pl.BlockSpec((Kd, Fo), lambda i: (0, 0)),
                  pl.BlockSpec((2, bm, _DP), lambda i: (0, i, 0))],
        out_specs=pl.BlockSpec((bm, Fo), lambda i: (i, 0)),
        out_shape=jax.ShapeDtypeStruct((B, Fo), F32))(x, w, degp)


def _gcn_mid_graph(p, y1, degp, b1, w2, bm):
    _, B, F1 = p.shape
    F2 = w2.shape[1]

    def kfn(p_ref, y_ref, d_ref, b_ref, w_ref, o_ref):
        dinv = _dinv_of(d_ref[...])
        xg = _elu(dinv * (p_ref[0] + p_ref[1] + y_ref[...]) + b_ref[...])
        o_ref[...] = dinv * jnp.dot(xg, w_ref[...], preferred_element_type=F32)

    return pl.pallas_call(
        kfn,
        grid=(B // bm,),
        in_specs=[pl.BlockSpec((2, bm, F1), lambda i: (0, i, 0)),
                  pl.BlockSpec((bm, F1), lambda i: (i, 0)),
                  pl.BlockSpec((2, bm, _DP), lambda i: (0, i, 0)),
                  pl.BlockSpec((1, F1), lambda i: (0, 0)),
                  pl.BlockSpec((F1, F2), lambda i: (0, 0))],
        out_specs=pl.BlockSpec((bm, F2), lambda i: (i, 0)),
        out_shape=jax.ShapeDtypeStruct((B, F2), F32))(p, y1, degp, b1, w2)


def _graph_head(p, y2, degp, b2, fcw, fcb):
    _, _, F2 = p.shape
    C = fcw.shape[1]

    def kfn(p_ref, y_ref, d_ref, b_ref, w_ref, fb_ref, o_ref):
        dinv = _dinv_of(d_ref[...])
        xg = _elu(dinv * (p_ref[0] + p_ref[1] + y_ref[...]) + b_ref[...])
        o_ref[...] = jnp.dot(xg, w_ref[...],
                             preferred_element_type=F32) + fb_ref[...]

    return pl.pallas_call(
        kfn,
        grid=(1,),
        in_specs=[pl.BlockSpec((2, _BATCH, F2), lambda i: (0, 0, 0)),
                  pl.BlockSpec((_BATCH, F2), lambda i: (0, 0)),
                  pl.BlockSpec((2, _BATCH, _DP), lambda i: (0, 0, 0)),
                  pl.BlockSpec((1, F2), lambda i: (0, 0)),
                  pl.BlockSpec((F2, C), lambda i: (0, 0)),
                  pl.BlockSpec((1, C), lambda i: (0, 0))],
        out_specs=pl.BlockSpec((_BATCH, C), lambda i: (0, 0)),
        out_shape=jax.ShapeDtypeStruct((_BATCH, C), F32))(
            p, y2, degp, b2, fcw, fcb)


def _tree_mid(p, y1, degp, b1, x1head, idxcol, wa, wb):
    N = y1.shape[0]

    def kfn(p_ref, y_ref, d_ref, b_ref, xh_ref, idx_ref, wa_ref, wb_ref, o_ref):
        dinv = _dinv_of(d_ref[...])
        xca = _elu(dinv * (p_ref[0] + p_ref[1] + y_ref[...]) + b_ref[...])
        cols = lax.broadcasted_iota(jnp.int32, (N, _BATCH), 1)
        oh = (cols == idx_ref[...]).astype(F32)
        xcb = _elu(jnp.dot(oh, xh_ref[...], preferred_element_type=F32))
        o_ref[...] = dinv * (
            jnp.dot(xca, wa_ref[...], preferred_element_type=F32)
            + jnp.dot(xcb, wb_ref[...], preferred_element_type=F32))

    return pl.pallas_call(
        kfn,
        out_shape=jax.ShapeDtypeStruct((N, _DP), F32))(
            p, y1, degp, b1, x1head, idxcol, wa, wb)


def _tree_post(p, y2, degp, b2, idxrow):
    N = y2.shape[0]

    def kfn(p_ref, y_ref, d_ref, b_ref, idx_ref, o_ref):
        dinv = _dinv_of(d_ref[...])
        xc2 = _elu(dinv * (p_ref[0] + p_ref[1] + y_ref[...]) + b_ref[...])
        rows = lax.broadcasted_iota(jnp.int32, (_BATCH, N), 0)
        oht = (rows == idx_ref[...]).astype(F32)
        seg = jnp.dot(oht, xc2, preferred_element_type=F32)
        cnt = jnp.sum(oht, axis=1, keepdims=True)
        o_ref[...] = seg / cnt

    return pl.pallas_call(
        kfn,
        out_shape=jax.ShapeDtypeStruct((_BATCH, _DP), F32))(
            p, y2, degp, b2, idxrow)


def _pad2(a, r, c):
    return jnp.pad(a, ((0, r - a.shape[0]), (0, c - a.shape[1])))


def _gates_T(W, kpad):
    wt = W.T.reshape(W.shape[1], 3, _H)
    wt = jnp.pad(wt, ((0, kpad - W.shape[1]), (0, 0), (0, _GP - _H)))
    return wt.reshape(kpad, _G3)


def _gates_b(b):
    return jnp.pad(b.reshape(3, _H), ((0, 0), (0, _GP - _H))).reshape(_G3)


def kernel(user_feats, graph_node_features, graph_edge_index,
           merged_tree_feature, merged_tree_edge_index, indices,
           emb_tree, emb_graph, h0_tree, h0_graph,
           ue_W1, ue_b1, ue_W2, ue_b2,
           gt_Wih0, gt_Whh0, gt_bih0, gt_bhh0,
           gt_Wih1, gt_Whh1, gt_bih1, gt_bhh1,
           gg_Wih0, gg_Whh0, gg_bih0, gg_bhh0,
           gg_Wih1, gg_Whh1, gg_bih1, gg_bhh1,
           tc1_W, tc1_b, tc2_W, tc2_b,
           gc1_W, gc1_b, gc2_W, gc2_b,
           fc_W, fc_b):
    i32 = jnp.int32
    tree_tok = merged_tree_feature.astype(i32).T.reshape(-1)
    graph_tok = graph_node_features.astype(i32).T.reshape(-1)
    embt_p = _pad2(emb_tree, _VOCAB, _DP)
    embg_p = _pad2(emb_graph, _VOCAB, _DP)
    g_src = graph_edge_index[0].astype(i32)
    g_dst = graph_edge_index[1].astype(i32)
    t_src = merged_tree_edge_index[1].astype(i32)
    t_dst = merged_tree_edge_index[0].astype(i32)
    idx_i = indices.astype(i32)
    h0t = jnp.pad(h0_tree, ((0, 0), (0, 0), (0, _GP - _H)))
    h0g = jnp.pad(h0_graph, ((0, 0), (0, 0), (0, _GP - _H)))

    wih0_t = _gates_T(gt_Wih0, _DP)
    whh0_t = _gates_T(gt_Whh0, _GP)
    wih1_t = _gates_T(gt_Wih1, _GP)
    whh1_t = _gates_T(gt_Whh1, _GP)
    wih0_g = _gates_T(gg_Wih0, _DP)
    whh0_g = _gates_T(gg_Whh0, _GP)
    wih1_g = _gates_T(gg_Wih1, _GP)
    whh1_g = _gates_T(gg_Whh1, _GP)
    bih0_t = _gates_b(gt_bih0)
    bhh0_t = _gates_b(gt_bhh0)
    bih1_t = _gates_b(gt_bih1)
    bhh1_t = _gates_b(gt_bhh1)
    bih0_g = _gates_b(gg_bih0)
    bhh0_g = _gates_b(gg_bhh0)
    bih1_g = _gates_b(gg_bih1)
    bhh1_g = _gates_b(gg_bhh1)

    tc1_Wp = _pad2(tc1_W, _GP, _DP)
    tc2_Wa = _pad2(tc2_W[:_H], _DP, _DP)
    tc2_Wb = _pad2(tc2_W[_H:], _GP, _DP)
    tc1_bp = _pad2(tc1_b[None], 1, _DP)
    tc2_bp = _pad2(tc2_b[None], 1, _DP)
    gc2_Wp = _pad2(gc2_W, _DP, _DP)
    gc2_bp = _pad2(gc2_b[None], 1, _DP)
    fc_Wp = _pad2(fc_W, _DP, fc_W.shape[1])

    degp_g = _sc_deg(g_dst, _N_GRAPH, _E_GRAPH)
    degp_t = _sc_deg(t_dst, _N_TREE, _E_TREE)
    xt = _sc_gather(embt_p, tree_tok, _SEQ * _N_TREE, _DP)
    xg = _sc_gather(embg_p, graph_tok, _SEQ * _N_GT, _DP)

    x1 = _gru2_tc(xt.reshape(_SEQ, _N_TREE, _DP), h0t[0], h0t[1],
                  wih0_t, whh0_t, wih1_t, whh1_t,
                  bih0_t, bhh0_t, bih1_t, bhh1_t, 512)
    hng = _gru2_tc(xg.reshape(_SEQ, _N_GT, _DP), h0g[0], h0g[1],
                   wih0_g, whh0_g, wih1_g, whh1_g,
                   bih0_g, bhh0_g, bih1_g, bhh1_g, 512)

    y1t = _gcn_pre(x1, tc1_Wp, degp_t, 2048)
    tp1 = _sc_agg(y1t, t_src, t_dst, _N_TREE, _DP, _E_TREE)
    y2t = _tree_mid(tp1, y1t, degp_t, tc1_bp, x1[:_BATCH],
                    idx_i[:, None], tc2_Wa, tc2_Wb)
    tp2 = _sc_agg(y2t, t_src, t_dst, _N_TREE, _DP, _E_TREE)
    temb = _tree_post(tp2, y2t, degp_t, tc2_bp, idx_i[None, :])

    ue = _ue_tc(user_feats, ue_W1, ue_b1, ue_W2, ue_b2)
    x_input = jnp.concatenate(
        [temb[:, :_H], ue, hng[_BATCH:, :_H]], axis=0)
    y1g = _gcn_pre(x_input, _pad2(gc1_W, gc1_W.shape[0], _DP), degp_g, 1024)
    gp1 = _sc_agg(y1g, g_src, g_dst, _N_GRAPH, _DP, _E_GRAPH)
    y2g = _gcn_mid_graph(gp1, y1g, degp_g, _pad2(gc1_b[None], 1, _DP), gc2_Wp, 1024)
    gp2 = _sc_agg(y2g, g_src, g_dst, _N_GRAPH, _DP, _E_GRAPH)
    out = _graph_head(gp2, y2g, degp_g, gc2_bp, fc_Wp, fc_b[None])
    return out


# speedup vs baseline: 6.9530x; 1.0015x over previous
import functools

import jax
import jax.numpy as jnp
from jax import lax
from jax.experimental import pallas as pl
from jax.experimental.pallas import tpu as pltpu
from jax.experimental.pallas import tpu_sc as plsc

F32 = jnp.float32

_N_USERS = 2048
_N_GT = 4096
_N_TREE = 2048
_VOCAB = 30000
_D = 100
_H = 100
_SEQ = 20
_E_GRAPH = 65536
_E_TREE = 2048
_BATCH = 32
_N_GRAPH = _N_GT + _N_USERS

_DP = 128
_GP = 128
_G3 = 3 * _GP

_NC = 2
_NS = 16
_NW = _NC * _NS


def _sc_mesh():
    return plsc.VectorSubcoreMesh(core_axis_name="c", subcore_axis_name="s",
                                  num_cores=_NC, num_subcores=_NS)


def _sc_gather(table, idx, B, Dp):
    bpw = B // _NW
    K = min(128, bpw)
    nch = bpw // K

    def body(table_hbm, idx_hbm, out_hbm, idx_v, rows_v, sem):
        c = lax.axis_index("c")
        s = lax.axis_index("s")
        base = (s * _NC + c) * bpw

        def step(j, carry):
            off = base + j * K
            pltpu.sync_copy(idx_hbm.at[pl.ds(off, K)], idx_v)
            pltpu.async_copy(table_hbm.at[idx_v], rows_v, sem).wait()
            pltpu.sync_copy(rows_v, out_hbm.at[pl.ds(off, K)])
            return carry

        lax.fori_loop(0, nch, step, 0)

    k = pl.kernel(
        body,
        out_type=jax.ShapeDtypeStruct((B, Dp), F32),
        mesh=_sc_mesh(),
        compiler_params=pltpu.CompilerParams(use_tc_tiling_on_sc=True),
        scratch_types=[pltpu.VMEM((K,), jnp.int32),
                       pltpu.VMEM((K, Dp), F32),
                       pltpu.SemaphoreType.DMA])
    return k(table, idx)


def _sc_agg(y, src, dst, N, F, E):
    epw = E // _NW
    K = min(128, epw)
    nch = epw // K
    rpt = N // _NS
    zeros = jnp.zeros((N, F), F32)

    def body(y_hbm, src_hbm, dst_hbm, z_hbm, out_hbm,
             si0, si1, di_v, rows0, rows1, acc_sh, sem0, sem1):
        c = lax.axis_index("c")
        s = lax.axis_index("s")
        base = (s * _NC + c) * epw
        zslc = pl.ds(s * rpt, rpt)
        pltpu.sync_copy(z_hbm.at[zslc], acc_sh.at[zslc])
        plsc.subcore_barrier()

        def start(j, si, rows, sem):
            off = base + j * K
            pltpu.sync_copy(src_hbm.at[pl.ds(off, K)], si)
            pltpu.async_copy(y_hbm.at[si], rows, sem)

        def finish(j, rows):
            off = base + j * K
            pltpu.sync_copy(dst_hbm.at[pl.ds(off, K)], di_v)
            pltpu.sync_copy(rows, acc_sh.at[di_v], add=True)

        if nch == 1:
            start(0, si0, rows0, sem0)
            pltpu.make_async_copy(y_hbm.at[si0], rows0, sem0).wait()
            finish(0, rows0)
        else:
            start(0, si0, rows0, sem0)

            def step2(k2, carry):
                j0 = 2 * k2
                pltpu.make_async_copy(y_hbm.at[si0], rows0, sem0).wait()
                start(jnp.minimum(j0 + 1, nch - 1), si1, rows1, sem1)
                finish(j0, rows0)
                pltpu.make_async_copy(y_hbm.at[si1], rows1, sem1).wait()
                start(jnp.minimum(j0 + 2, nch - 1), si0, rows0, sem0)
                finish(j0 + 1, rows1)
                return carry

            lax.fori_loop(0, nch // 2, step2, 0)
            pltpu.make_async_copy(y_hbm.at[si0], rows0, sem0).wait()
        plsc.subcore_barrier()
        pltpu.sync_copy(acc_sh.at[zslc], out_hbm.at[c, zslc])

    k = pl.kernel(
        body,
        out_type=jax.ShapeDtypeStruct((_NC, N, F), F32),
        mesh=_sc_mesh(),
        scratch_types=[pltpu.VMEM((K,), jnp.int32),
                       pltpu.VMEM((K,), jnp.int32),
                       pltpu.VMEM((K,), jnp.int32),
                       pltpu.VMEM((K, F), F32),
                       pltpu.VMEM((K, F), F32),
                       pltpu.VMEM_SHARED((N, F), F32),
                       pltpu.SemaphoreType.DMA,
                       pltpu.SemaphoreType.DMA])
    return k(y, src, dst, zeros)


def _sc_deg(dst, N, E):
    epw = E // _NW
    K = min(128, epw)
    nch = epw // K
    rpt = N // _NS
    ones = jnp.ones((K, _DP), F32)
    zeros = jnp.zeros((N, _DP), F32)

    def body(ones_hbm, z_hbm, dst_hbm, out_hbm, di_v, ones_v, acc_sh):
        c = lax.axis_index("c")
        s = lax.axis_index("s")
        base = (s * _NC + c) * epw
        pltpu.sync_copy(ones_hbm, ones_v)
        zslc = pl.ds(s * rpt, rpt)
        pltpu.sync_copy(z_hbm.at[zslc], acc_sh.at[zslc])
        plsc.subcore_barrier()

        def step(j, carry):
            off = base + j * K
            pltpu.sync_copy(dst_hbm.at[pl.ds(off, K)], di_v)
            pltpu.sync_copy(ones_v, acc_sh.at[di_v], add=True)
            return carry

        lax.fori_loop(0, nch, step, 0)
        plsc.subcore_barrier()
        pltpu.sync_copy(acc_sh.at[zslc], out_hbm.at[c, zslc])

    k = pl.kernel(
        body,
        out_type=jax.ShapeDtypeStruct((_NC, N, _DP), F32),
        mesh=_sc_mesh(),
        scratch_types=[pltpu.VMEM((K,), jnp.int32),
                       pltpu.VMEM((K, _DP), F32),
                       pltpu.VMEM_SHARED((N, _DP), F32)])
    return k(ones, zeros, dst)


def _gru2_tc(x, h01, h02, wih0, whh0, wih1, whh1,
             bih0, bhh0, bih1, bhh1, nb):
    S, N, Dx = x.shape

    def kfn(x_ref, h01_ref, h02_ref, wi0_ref, w0_ref, w1_ref, w2_ref,
            bi0_ref, b0_ref, b1_ref, b2_ref, o_ref):
        h1 = h01_ref[...]
        h2 = h02_ref[...]
        wi0 = wi0_ref[...]
        w0 = w0_ref[...]
        w1 = w1_ref[...]
        w2 = w2_ref[...]
        bi0 = bi0_ref[...]
        b0 = b0_ref[...]
        b1 = b1_ref[...]
        b2 = b2_ref[...]

        def gate(gi_t, gh_t, h):
            r = jax.nn.sigmoid(gi_t[:, 0:_GP] + gh_t[:, 0:_GP])
            z = jax.nn.sigmoid(gi_t[:, _GP:2 * _GP] + gh_t[:, _GP:2 * _GP])
            n = jnp.tanh(gi_t[:, 2 * _GP:] + r * gh_t[:, 2 * _GP:])
            return (1.0 - z) * n + z * h

        for t in range(S):
            gi1 = jnp.dot(x_ref[t], wi0, preferred_element_type=F32) + bi0
            gh1 = jnp.dot(h1, w0, preferred_element_type=F32) + b0
            h1 = gate(gi1, gh1, h1)
            gi2 = jnp.dot(h1, w1, preferred_element_type=F32) + b1
            gh2 = jnp.dot(h2, w2, preferred_element_type=F32) + b2
            h2 = gate(gi2, gh2, h2)
        o_ref[...] = h2

    wispec = pl.BlockSpec((Dx, _G3), lambda i: (0, 0))
    wspec = pl.BlockSpec((_GP, _G3), lambda i: (0, 0))
    bspec = pl.BlockSpec((1, _G3), lambda i: (0, 0))
    hspec = pl.BlockSpec((nb, _GP), lambda i: (i, 0))
    return pl.pallas_call(
        kfn,
        grid=(N // nb,),
        in_specs=[pl.BlockSpec((S, nb, Dx), lambda i: (0, i, 0)),
                  hspec, hspec, wispec, wspec, wspec, wspec,
                  bspec, bspec, bspec, bspec],
        out_specs=hspec,
        out_shape=jax.ShapeDtypeStruct((N, _GP), F32))(
            x, h01, h02, wih0, whh0, wih1, whh1,
            bih0[None], bhh0[None], bih1[None], bhh1[None])


def _ue_tc(uf, w1, b1, w2, b2):
    def kfn(x_ref, w1_ref, b1_ref, w2_ref, b2_ref, o_ref):
        h = jnp.maximum(jnp.dot(x_ref[...], w1_ref[...],
                                preferred_element_type=F32) + b1_ref[...], 0.0)
        o_ref[...] = jnp.dot(h, w2_ref[...],
                             preferred_element_type=F32) + b2_ref[...]

    return pl.pallas_call(
        kfn,
        out_shape=jax.ShapeDtypeStruct((uf.shape[0], w2.shape[1]), F32))(
            uf, w1, b1[None], w2, b2[None])


def _dinv_of(d):
    return lax.rsqrt(d[0, :, 0:1] + d[1, :, 0:1] + 1.0)


def _elu(x):
    return jnp.where(x > 0, x, jnp.exp(jnp.minimum(x, 0.0)) - 1.0)


def _gcn_pre(x, w, degp, bm):
    B, Kd = x.shape
    Fo = w.shape[1]

    def kfn(x_ref, w_ref, d_ref, o_ref):
        o_ref[...] = _dinv_of(d_ref[...]) * jnp.dot(
            x_ref[...], w_ref[...], preferred_element_type=F32)

    return pl.pallas_call(
        kfn,
        grid=(B // bm,),
        in_specs=[pl.BlockSpec((bm, Kd), lambda i: (i, 0)),
                  pl.BlockSpec((Kd, Fo), lambda i: (0, 0)),
                  pl.BlockSpec((2, bm, _DP), lambda i: (0, i, 0))],
        out_specs=pl.BlockSpec((bm, Fo), lambda i: (i, 0)),
        out_shape=jax.ShapeDtypeStruct((B, Fo), F32))(x, w, degp)


def _gcn_mid_graph(p, y1, degp, b1, w2, bm):
    _, B, F1 = p.shape
    F2 = w2.shape[1]

    def kfn(p_ref, y_ref, d_ref, b_ref, w_ref, o_ref):
        dinv = _dinv_of(d_ref[...])
        xg = _elu(dinv * (p_ref[0] + p_ref[1] + y_ref[...]) + b_ref[...])
        o_ref[...] = dinv * jnp.dot(xg, w_ref[...], preferred_element_type=F32)

    return pl.pallas_call(
        kfn,
        grid=(B // bm,),
        in_specs=[pl.BlockSpec((2, bm, F1), lambda i: (0, i, 0)),
                  pl.BlockSpec((bm, F1), lambda i: (i, 0)),
                  pl.BlockSpec((2, bm, _DP), lambda i: (0, i, 0)),
                  pl.BlockSpec((1, F1), lambda i: (0, 0)),
                  pl.BlockSpec((F1, F2), lambda i: (0, 0))],
        out_specs=pl.BlockSpec((bm, F2), lambda i: (i, 0)),
        out_shape=jax.ShapeDtypeStruct((B, F2), F32))(p, y1, degp, b1, w2)


def _graph_head(p, y2, degp, b2, fcw, fcb):
    _, _, F2 = p.shape
    C = fcw.shape[1]

    def kfn(p_ref, y_ref, d_ref, b_ref, w_ref, fb_ref, o_ref):
        dinv = _dinv_of(d_ref[...])
        xg = _elu(dinv * (p_ref[0] + p_ref[1] + y_ref[...]) + b_ref[...])
        o_ref[...] = jnp.dot(xg, w_ref[...],
                             preferred_element_type=F32) + fb_ref[...]

    return pl.pallas_call(
        kfn,
        grid=(1,),
        in_specs=[pl.BlockSpec((2, _BATCH, F2), lambda i: (0, 0, 0)),
                  pl.BlockSpec((_BATCH, F2), lambda i: (0, 0)),
                  pl.BlockSpec((2, _BATCH, _DP), lambda i: (0, 0, 0)),
                  pl.BlockSpec((1, F2), lambda i: (0, 0)),
                  pl.BlockSpec((F2, C), lambda i: (0, 0)),
                  pl.BlockSpec((1, C), lambda i: (0, 0))],
        out_specs=pl.BlockSpec((_BATCH, C), lambda i: (0, 0)),
        out_shape=jax.ShapeDtypeStruct((_BATCH, C), F32))(
            p, y2, degp, b2, fcw, fcb)


def _tree_mid(p, y1, degp, b1, x1head, idxcol, wa, wb):
    N = y1.shape[0]

    def kfn(p_ref, y_ref, d_ref, b_ref, xh_ref, idx_ref, wa_ref, wb_ref, o_ref):
        dinv = _dinv_of(d_ref[...])
        xca = _elu(dinv * (p_ref[0] + p_ref[1] + y_ref[...]) + b_ref[...])
        cols = lax.broadcasted_iota(jnp.int32, (N, _BATCH), 1)
        oh = (cols == idx_ref[...]).astype(F32)
        xcb = _elu(jnp.dot(oh, xh_ref[...], preferred_element_type=F32))
        o_ref[...] = dinv * (
            jnp.dot(xca, wa_ref[...], preferred_element_type=F32)
            + jnp.dot(xcb, wb_ref[...], preferred_element_type=F32))

    return pl.pallas_call(
        kfn,
        out_shape=jax.ShapeDtypeStruct((N, _DP), F32))(
            p, y1, degp, b1, x1head, idxcol, wa, wb)


def _tree_post(p, y2, degp, b2, idxrow):
    N = y2.shape[0]

    def kfn(p_ref, y_ref, d_ref, b_ref, idx_ref, o_ref):
        dinv = _dinv_of(d_ref[...])
        xc2 = _elu(dinv * (p_ref[0] + p_ref[1] + y_ref[...]) + b_ref[...])
        rows = lax.broadcasted_iota(jnp.int32, (_BATCH, N), 0)
        oht = (rows == idx_ref[...]).astype(F32)
        seg = jnp.dot(oht, xc2, preferred_element_type=F32)
        cnt = jnp.sum(oht, axis=1, keepdims=True)
        o_ref[...] = seg / cnt

    return pl.pallas_call(
        kfn,
        out_shape=jax.ShapeDtypeStruct((_BATCH, _DP), F32))(
            p, y2, degp, b2, idxrow)


def _pad2(a, r, c):
    return jnp.pad(a, ((0, r - a.shape[0]), (0, c - a.shape[1])))


def _gates_T(W, kpad):
    wt = W.T.reshape(W.shape[1], 3, _H)
    wt = jnp.pad(wt, ((0, kpad - W.shape[1]), (0, 0), (0, _GP - _H)))
    return wt.reshape(kpad, _G3)


def _gates_b(b):
    return jnp.pad(b.reshape(3, _H), ((0, 0), (0, _GP - _H))).reshape(_G3)


def kernel(user_feats, graph_node_features, graph_edge_index,
           merged_tree_feature, merged_tree_edge_index, indices,
           emb_tree, emb_graph, h0_tree, h0_graph,
           ue_W1, ue_b1, ue_W2, ue_b2,
           gt_Wih0, gt_Whh0, gt_bih0, gt_bhh0,
           gt_Wih1, gt_Whh1, gt_bih1, gt_bhh1,
           gg_Wih0, gg_Whh0, gg_bih0, gg_bhh0,
           gg_Wih1, gg_Whh1, gg_bih1, gg_bhh1,
           tc1_W, tc1_b, tc2_W, tc2_b,
           gc1_W, gc1_b, gc2_W, gc2_b,
           fc_W, fc_b):
    i32 = jnp.int32
    tree_tok = merged_tree_feature.astype(i32).T.reshape(-1)
    graph_tok = graph_node_features.astype(i32).T.reshape(-1)
    embt_p = _pad2(emb_tree, _VOCAB, _DP)
    embg_p = _pad2(emb_graph, _VOCAB, _DP)
    g_src = graph_edge_index[0].astype(i32)
    g_dst = graph_edge_index[1].astype(i32)
    t_src = merged_tree_edge_index[1].astype(i32)
    t_dst = merged_tree_edge_index[0].astype(i32)
    idx_i = indices.astype(i32)
    h0t = jnp.pad(h0_tree, ((0, 0), (0, 0), (0, _GP - _H)))
    h0g = jnp.pad(h0_graph, ((0, 0), (0, 0), (0, _GP - _H)))

    wih0_t = _gates_T(gt_Wih0, _DP)
    whh0_t = _gates_T(gt_Whh0, _GP)
    wih1_t = _gates_T(gt_Wih1, _GP)
    whh1_t = _gates_T(gt_Whh1, _GP)
    wih0_g = _gates_T(gg_Wih0, _DP)
    whh0_g = _gates_T(gg_Whh0, _GP)
    wih1_g = _gates_T(gg_Wih1, _GP)
    whh1_g = _gates_T(gg_Whh1, _GP)
    bih0_t = _gates_b(gt_bih0)
    bhh0_t = _gates_b(gt_bhh0)
    bih1_t = _gates_b(gt_bih1)
    bhh1_t = _gates_b(gt_bhh1)
    bih0_g = _gates_b(gg_bih0)
    bhh0_g = _gates_b(gg_bhh0)
    bih1_g = _gates_b(gg_bih1)
    bhh1_g = _gates_b(gg_bhh1)

    tc1_Wp = _pad2(tc1_W, _GP, _DP)
    tc2_Wa = _pad2(tc2_W[:_H], _DP, _DP)
    tc2_Wb = _pad2(tc2_W[_H:], _GP, _DP)
    tc1_bp = _pad2(tc1_b[None], 1, _DP)
    tc2_bp = _pad2(tc2_b[None], 1, _DP)
    gc2_Wp = _pad2(gc2_W, _DP, _DP)
    gc2_bp = _pad2(gc2_b[None], 1, _DP)
    fc_Wp = _pad2(fc_W, _DP, fc_W.shape[1])

    degp_g = _sc_deg(g_dst, _N_GRAPH, _E_GRAPH)
    degp_t = _sc_deg(t_dst, _N_TREE, _E_TREE)
    xt = _sc_gather(embt_p, tree_tok, _SEQ * _N_TREE, _DP)
    xg = _sc_gather(embg_p, graph_tok, _SEQ * _N_GT, _DP)

    x1 = _gru2_tc(xt.reshape(_SEQ, _N_TREE, _DP), h0t[0], h0t[1],
                  wih0_t, whh0_t, wih1_t, whh1_t,
                  bih0_t, bhh0_t, bih1_t, bhh1_t, 512)
    hng = _gru2_tc(xg.reshape(_SEQ, _N_GT, _DP), h0g[0], h0g[1],
                   wih0_g, whh0_g, wih1_g, whh1_g,
                   bih0_g, bhh0_g, bih1_g, bhh1_g, 512)

    y1t = _gcn_pre(x1, tc1_Wp, degp_t, 2048)
    tp1 = _sc_agg(y1t, t_src, t_dst, _N_TREE, _DP, _E_TREE)
    y2t = _tree_mid(tp1, y1t, degp_t, tc1_bp, x1[:_BATCH],
                    idx_i[:, None], tc2_Wa, tc2_Wb)
    tp2 = _sc_agg(y2t, t_src, t_dst, _N_TREE, _DP, _E_TREE)
    temb = _tree_post(tp2, y2t, degp_t, tc2_bp, idx_i[None, :])

    ue = _ue_tc(user_feats, ue_W1, ue_b1, ue_W2, ue_b2)
    x_input = jnp.concatenate(
        [temb[:, :_H], ue, hng[_BATCH:, :_H]], axis=0)
    y1g = _gcn_pre(x_input, _pad2(gc1_W, gc1_W.shape[0], _DP), degp_g, 1024)
    gp1 = _sc_agg(y1g, g_src, g_dst, _N_GRAPH, _DP, _E_GRAPH)
    y2g = _gcn_mid_graph(gp1, y1g, degp_g, _pad2(gc1_b[None], 1, _DP), gc2_Wp, 1024)
    gp2 = _sc_agg(y2g, g_src, g_dst, _N_GRAPH, _DP, _E_GRAPH)
    out = _graph_head(gp2, y2g, degp_g, gc2_bp, fc_Wp, fc_b[None])
    return out


# speedup vs baseline: 7.7323x; 1.1121x over previous
import functools

import jax
import jax.numpy as jnp
from jax import lax
from jax.experimental import pallas as pl
from jax.experimental.pallas import tpu as pltpu
from jax.experimental.pallas import tpu_sc as plsc

F32 = jnp.float32

_N_USERS = 2048
_N_GT = 4096
_N_TREE = 2048
_VOCAB = 30000
_D = 100
_H = 100
_SEQ = 20
_E_GRAPH = 65536
_E_TREE = 2048
_BATCH = 32
_N_GRAPH = _N_GT + _N_USERS

_DP = 128
_GP = 128
_G3 = 3 * _GP

_NC = 2
_NS = 16
_NW = _NC * _NS


def _sc_mesh():
    return plsc.VectorSubcoreMesh(core_axis_name="c", subcore_axis_name="s",
                                  num_cores=_NC, num_subcores=_NS)


def _sc_gather(table, idx, B, Dp):
    bpw = B // _NW
    K = min(128, bpw)
    nch = bpw // K

    def body(table_hbm, idx_hbm, out_hbm, idx_v, rows_v, sem):
        c = lax.axis_index("c")
        s = lax.axis_index("s")
        base = (s * _NC + c) * bpw

        def step(j, carry):
            off = base + j * K
            pltpu.sync_copy(idx_hbm.at[pl.ds(off, K)], idx_v)
            pltpu.async_copy(table_hbm.at[idx_v], rows_v, sem).wait()
            pltpu.sync_copy(rows_v, out_hbm.at[pl.ds(off, K)])
            return carry

        lax.fori_loop(0, nch, step, 0)

    k = pl.kernel(
        body,
        out_type=jax.ShapeDtypeStruct((B, Dp), F32),
        mesh=_sc_mesh(),
        compiler_params=pltpu.CompilerParams(use_tc_tiling_on_sc=True),
        scratch_types=[pltpu.VMEM((K,), jnp.int32),
                       pltpu.VMEM((K, Dp), F32),
                       pltpu.SemaphoreType.DMA])
    return k(table, idx)


def _sc_agg(y, src, dst, N, F, E):
    epw = E // _NW
    K = min(128, epw)
    nch = epw // K
    rpt = N // _NS
    zeros = jnp.zeros((N, F), F32)

    def body(y_hbm, src_hbm, dst_hbm, z_hbm, out_hbm,
             si0, si1, di_v, rows0, rows1, acc_sh, sem0, sem1):
        c = lax.axis_index("c")
        s = lax.axis_index("s")
        base = (s * _NC + c) * epw
        zslc = pl.ds(s * rpt, rpt)
        pltpu.sync_copy(z_hbm.at[zslc], acc_sh.at[zslc])
        plsc.subcore_barrier()

        def start(j, si, rows, sem):
            off = base + j * K
            pltpu.sync_copy(src_hbm.at[pl.ds(off, K)], si)
            pltpu.async_copy(y_hbm.at[si], rows, sem)

        def finish(j, rows):
            off = base + j * K
            pltpu.sync_copy(dst_hbm.at[pl.ds(off, K)], di_v)
            pltpu.sync_copy(rows, acc_sh.at[di_v], add=True)

        if nch == 1:
            start(0, si0, rows0, sem0)
            pltpu.make_async_copy(y_hbm.at[si0], rows0, sem0).wait()
            finish(0, rows0)
        else:
            start(0, si0, rows0, sem0)

            def step2(k2, carry):
                j0 = 2 * k2
                pltpu.make_async_copy(y_hbm.at[si0], rows0, sem0).wait()
                start(jnp.minimum(j0 + 1, nch - 1), si1, rows1, sem1)
                finish(j0, rows0)
                pltpu.make_async_copy(y_hbm.at[si1], rows1, sem1).wait()
                start(jnp.minimum(j0 + 2, nch - 1), si0, rows0, sem0)
                finish(j0 + 1, rows1)
                return carry

            lax.fori_loop(0, nch // 2, step2, 0)
            pltpu.make_async_copy(y_hbm.at[si0], rows0, sem0).wait()
        plsc.subcore_barrier()
        pltpu.sync_copy(acc_sh.at[zslc], out_hbm.at[c, zslc])

    k = pl.kernel(
        body,
        out_type=jax.ShapeDtypeStruct((_NC, N, F), F32),
        mesh=_sc_mesh(),
        scratch_types=[pltpu.VMEM((K,), jnp.int32),
                       pltpu.VMEM((K,), jnp.int32),
                       pltpu.VMEM((K,), jnp.int32),
                       pltpu.VMEM((K, F), F32),
                       pltpu.VMEM((K, F), F32),
                       pltpu.VMEM_SHARED((N, F), F32),
                       pltpu.SemaphoreType.DMA,
                       pltpu.SemaphoreType.DMA])
    return k(y, src, dst, zeros)


def _sc_deg(dst, N, E):
    epw = E // _NW
    K = min(128, epw)
    nch = epw // K
    rpt = N // _NS
    ones = jnp.ones((K, _DP), F32)
    zeros = jnp.zeros((N, _DP), F32)

    def body(ones_hbm, z_hbm, dst_hbm, out_hbm, di_v, ones_v, acc_sh):
        c = lax.axis_index("c")
        s = lax.axis_index("s")
        base = (s * _NC + c) * epw
        pltpu.sync_copy(ones_hbm, ones_v)
        zslc = pl.ds(s * rpt, rpt)
        pltpu.sync_copy(z_hbm.at[zslc], acc_sh.at[zslc])
        plsc.subcore_barrier()

        def step(j, carry):
            off = base + j * K
            pltpu.sync_copy(dst_hbm.at[pl.ds(off, K)], di_v)
            pltpu.sync_copy(ones_v, acc_sh.at[di_v], add=True)
            return carry

        lax.fori_loop(0, nch, step, 0)
        plsc.subcore_barrier()
        pltpu.sync_copy(acc_sh.at[zslc], out_hbm.at[c, zslc])

    k = pl.kernel(
        body,
        out_type=jax.ShapeDtypeStruct((_NC, N, _DP), F32),
        mesh=_sc_mesh(),
        scratch_types=[pltpu.VMEM((K,), jnp.int32),
                       pltpu.VMEM((K, _DP), F32),
                       pltpu.VMEM_SHARED((N, _DP), F32)])
    return k(ones, zeros, dst)


def _pad_cols_tc(a, cols_out, bm):
    R, C = a.shape

    def kfn(a_ref, o_ref):
        o_ref[...] = jnp.concatenate(
            [a_ref[...], jnp.zeros((bm, cols_out - C), F32)], axis=1)

    return pl.pallas_call(
        kfn,
        grid=(R // bm,),
        in_specs=[pl.BlockSpec((bm, C), lambda i: (i, 0))],
        out_specs=pl.BlockSpec((bm, cols_out), lambda i: (i, 0)),
        out_shape=jax.ShapeDtypeStruct((R, cols_out), F32))(a)

def _gru2_tc(x, h01, h02, wih0, whh0, wih1, whh1,
             bih0, bhh0, bih1, bhh1, nb):
    S, N, Dx = x.shape

    def kfn(x_ref, h01_ref, h02_ref, wi0_ref, w0_ref, w1_ref, w2_ref,
            bi0_ref, b0_ref, b1_ref, b2_ref, o_ref):
        h1 = h01_ref[...]
        h2 = h02_ref[...]
        wi0 = wi0_ref[...]
        w0 = w0_ref[...]
        w1 = w1_ref[...]
        w2 = w2_ref[...]
        bi0 = bi0_ref[...]
        b0 = b0_ref[...]
        b1 = b1_ref[...]
        b2 = b2_ref[...]

        def gate(gi_t, gh_t, h):
            r = jax.nn.sigmoid(gi_t[:, 0:_GP] + gh_t[:, 0:_GP])
            z = jax.nn.sigmoid(gi_t[:, _GP:2 * _GP] + gh_t[:, _GP:2 * _GP])
            n = jnp.tanh(gi_t[:, 2 * _GP:] + r * gh_t[:, 2 * _GP:])
            return (1.0 - z) * n + z * h

        for t in range(S):
            gi1 = jnp.dot(x_ref[t], wi0, preferred_element_type=F32) + bi0
            gh1 = jnp.dot(h1, w0, preferred_element_type=F32) + b0
            h1 = gate(gi1, gh1, h1)
            gi2 = jnp.dot(h1, w1, preferred_element_type=F32) + b1
            gh2 = jnp.dot(h2, w2, preferred_element_type=F32) + b2
            h2 = gate(gi2, gh2, h2)
        o_ref[...] = h2

    wispec = pl.BlockSpec((Dx, _G3), lambda i: (0, 0))
    wspec = pl.BlockSpec((_GP, _G3), lambda i: (0, 0))
    bspec = pl.BlockSpec((1, _G3), lambda i: (0, 0))
    hspec = pl.BlockSpec((nb, _GP), lambda i: (i, 0))
    return pl.pallas_call(
        kfn,
        grid=(N // nb,),
        in_specs=[pl.BlockSpec((S, nb, Dx), lambda i: (0, i, 0)),
                  hspec, hspec, wispec, wspec, wspec, wspec,
                  bspec, bspec, bspec, bspec],
        out_specs=hspec,
        out_shape=jax.ShapeDtypeStruct((N, _GP), F32))(
            x, h01, h02, wih0, whh0, wih1, whh1,
            bih0[None], bhh0[None], bih1[None], bhh1[None])


def _ue_tc(uf, w1, b1, w2, b2):
    def kfn(x_ref, w1_ref, b1_ref, w2_ref, b2_ref, o_ref):
        h = jnp.maximum(jnp.dot(x_ref[...], w1_ref[...],
                                preferred_element_type=F32) + b1_ref[...], 0.0)
        o_ref[...] = jnp.dot(h, w2_ref[...],
                             preferred_element_type=F32) + b2_ref[...]

    return pl.pallas_call(
        kfn,
        out_shape=jax.ShapeDtypeStruct((uf.shape[0], w2.shape[1]), F32))(
            uf, w1, b1[None], w2, b2[None])


def _dinv_of(d):
    return lax.rsqrt(d[0, :, 0:1] + d[1, :, 0:1] + 1.0)


def _elu(x):
    return jnp.where(x > 0, x, jnp.exp(jnp.minimum(x, 0.0)) - 1.0)


def _gcn_pre(x, w, degp, bm):
    B, Kd = x.shape
    Fo = w.shape[1]

    def kfn(x_ref, w_ref, d_ref, o_ref):
        o_ref[...] = _dinv_of(d_ref[...]) * jnp.dot(
            x_ref[...], w_ref[...], preferred_element_type=F32)

    return pl.pallas_call(
        kfn,
        grid=(B // bm,),
        in_specs=[pl.BlockSpec((bm, Kd), lambda i: (i, 0)),
                  pl.BlockSpec((Kd, Fo), lambda i: (0, 0)),
                  pl.BlockSpec((2, bm, _DP), lambda i: (0, i, 0))],
        out_specs=pl.BlockSpec((bm, Fo), lambda i: (i, 0)),
        out_shape=jax.ShapeDtypeStruct((B, Fo), F32))(x, w, degp)


def _gcn_mid_graph(p, y1, degp, b1, w2, bm):
    _, B, F1 = p.shape
    F2 = w2.shape[1]

    def kfn(p_ref, y_ref, d_ref, b_ref, w_ref, o_ref):
        dinv = _dinv_of(d_ref[...])
        xg = _elu(dinv * (p_ref[0] + p_ref[1] + y_ref[...]) + b_ref[...])
        o_ref[...] = dinv * jnp.dot(xg, w_ref[...], preferred_element_type=F32)

    return pl.pallas_call(
        kfn,
        grid=(B // bm,),
        in_specs=[pl.BlockSpec((2, bm, F1), lambda i: (0, i, 0)),
                  pl.BlockSpec((bm, F1), lambda i: (i, 0)),
                  pl.BlockSpec((2, bm, _DP), lambda i: (0, i, 0)),
                  pl.BlockSpec((1, F1), lambda i: (0, 0)),
                  pl.BlockSpec((F1, F2), lambda i: (0, 0))],
        out_specs=pl.BlockSpec((bm, F2), lambda i: (i, 0)),
        out_shape=jax.ShapeDtypeStruct((B, F2), F32))(p, y1, degp, b1, w2)


def _graph_head(p, y2, degp, b2, fcw, fcb):
    _, _, F2 = p.shape
    C = fcw.shape[1]

    def kfn(p_ref, y_ref, d_ref, b_ref, w_ref, fb_ref, o_ref):
        dinv = _dinv_of(d_ref[...])
        xg = _elu(dinv * (p_ref[0] + p_ref[1] + y_ref[...]) + b_ref[...])
        o_ref[...] = jnp.dot(xg, w_ref[...],
                             preferred_element_type=F32) + fb_ref[...]

    return pl.pallas_call(
        kfn,
        grid=(1,),
        in_specs=[pl.BlockSpec((2, _BATCH, F2), lambda i: (0, 0, 0)),
                  pl.BlockSpec((_BATCH, F2), lambda i: (0, 0)),
                  pl.BlockSpec((2, _BATCH, _DP), lambda i: (0, 0, 0)),
                  pl.BlockSpec((1, F2), lambda i: (0, 0)),
                  pl.BlockSpec((F2, C), lambda i: (0, 0)),
                  pl.BlockSpec((1, C), lambda i: (0, 0))],
        out_specs=pl.BlockSpec((_BATCH, C), lambda i: (0, 0)),
        out_shape=jax.ShapeDtypeStruct((_BATCH, C), F32))(
            p, y2, degp, b2, fcw, fcb)


def _tree_mid(p, y1, degp, b1, x1head, idxcol, wa, wb):
    N = y1.shape[0]

    def kfn(p_ref, y_ref, d_ref, b_ref, xh_ref, idx_ref, wa_ref, wb_ref, o_ref):
        dinv = _dinv_of(d_ref[...])
        xca = _elu(dinv * (p_ref[0] + p_ref[1] + y_ref[...]) + b_ref[...])
        cols = lax.broadcasted_iota(jnp.int32, (N, _BATCH), 1)
        oh = (cols == idx_ref[...]).astype(F32)
        xcb = _elu(jnp.dot(oh, xh_ref[...], preferred_element_type=F32))
        o_ref[...] = dinv * (
            jnp.dot(xca, wa_ref[...], preferred_element_type=F32)
            + jnp.dot(xcb, wb_ref[...], preferred_element_type=F32))

    return pl.pallas_call(
        kfn,
        out_shape=jax.ShapeDtypeStruct((N, _DP), F32))(
            p, y1, degp, b1, x1head, idxcol, wa, wb)


def _tree_post(p, y2, degp, b2, idxrow):
    N = y2.shape[0]

    def kfn(p_ref, y_ref, d_ref, b_ref, idx_ref, o_ref):
        dinv = _dinv_of(d_ref[...])
        xc2 = _elu(dinv * (p_ref[0] + p_ref[1] + y_ref[...]) + b_ref[...])
        rows = lax.broadcasted_iota(jnp.int32, (_BATCH, N), 0)
        oht = (rows == idx_ref[...]).astype(F32)
        seg = jnp.dot(oht, xc2, preferred_element_type=F32)
        cnt = jnp.sum(oht, axis=1, keepdims=True)
        o_ref[...] = seg / cnt

    return pl.pallas_call(
        kfn,
        out_shape=jax.ShapeDtypeStruct((_BATCH, _DP), F32))(
            p, y2, degp, b2, idxrow)


def _pad2(a, r, c):
    return jnp.pad(a, ((0, r - a.shape[0]), (0, c - a.shape[1])))


def _gates_T(W, kpad):
    wt = W.T.reshape(W.shape[1], 3, _H)
    wt = jnp.pad(wt, ((0, kpad - W.shape[1]), (0, 0), (0, _GP - _H)))
    return wt.reshape(kpad, _G3)


def _gates_b(b):
    return jnp.pad(b.reshape(3, _H), ((0, 0), (0, _GP - _H))).reshape(_G3)


def kernel(user_feats, graph_node_features, graph_edge_index,
           merged_tree_feature, merged_tree_edge_index, indices,
           emb_tree, emb_graph, h0_tree, h0_graph,
           ue_W1, ue_b1, ue_W2, ue_b2,
           gt_Wih0, gt_Whh0, gt_bih0, gt_bhh0,
           gt_Wih1, gt_Whh1, gt_bih1, gt_bhh1,
           gg_Wih0, gg_Whh0, gg_bih0, gg_bhh0,
           gg_Wih1, gg_Whh1, gg_bih1, gg_bhh1,
           tc1_W, tc1_b, tc2_W, tc2_b,
           gc1_W, gc1_b, gc2_W, gc2_b,
           fc_W, fc_b):
    i32 = jnp.int32
    tree_tok = merged_tree_feature.astype(i32).T.reshape(-1)
    graph_tok = graph_node_features.astype(i32).T.reshape(-1)
    embt_p = _pad_cols_tc(emb_tree, _DP, 3000)
    embg_p = _pad_cols_tc(emb_graph, _DP, 3000)
    g_src = graph_edge_index[0].astype(i32)
    g_dst = graph_edge_index[1].astype(i32)
    t_src = merged_tree_edge_index[1].astype(i32)
    t_dst = merged_tree_edge_index[0].astype(i32)
    idx_i = indices.astype(i32)
    h0t = jnp.pad(h0_tree, ((0, 0), (0, 0), (0, _GP - _H)))
    h0g = jnp.pad(h0_graph, ((0, 0), (0, 0), (0, _GP - _H)))

    wih0_t = _gates_T(gt_Wih0, _DP)
    whh0_t = _gates_T(gt_Whh0, _GP)
    wih1_t = _gates_T(gt_Wih1, _GP)
    whh1_t = _gates_T(gt_Whh1, _GP)
    wih0_g = _gates_T(gg_Wih0, _DP)
    whh0_g = _gates_T(gg_Whh0, _GP)
    wih1_g = _gates_T(gg_Wih1, _GP)
    whh1_g = _gates_T(gg_Whh1, _GP)
    bih0_t = _gates_b(gt_bih0)
    bhh0_t = _gates_b(gt_bhh0)
    bih1_t = _gates_b(gt_bih1)
    bhh1_t = _gates_b(gt_bhh1)
    bih0_g = _gates_b(gg_bih0)
    bhh0_g = _gates_b(gg_bhh0)
    bih1_g = _gates_b(gg_bih1)
    bhh1_g = _gates_b(gg_bhh1)

    tc1_Wp = _pad2(tc1_W, _GP, _DP)
    tc2_Wa = _pad2(tc2_W[:_H], _DP, _DP)
    tc2_Wb = _pad2(tc2_W[_H:], _GP, _DP)
    tc1_bp = _pad2(tc1_b[None], 1, _DP)
    tc2_bp = _pad2(tc2_b[None], 1, _DP)
    gc2_Wp = _pad2(gc2_W, _DP, _DP)
    gc2_bp = _pad2(gc2_b[None], 1, _DP)
    fc_Wp = _pad2(fc_W, _DP, fc_W.shape[1])

    degp_g = _sc_deg(g_dst, _N_GRAPH, _E_GRAPH)
    degp_t = _sc_deg(t_dst, _N_TREE, _E_TREE)
    xt = _sc_gather(embt_p, tree_tok, _SEQ * _N_TREE, _DP)
    xg = _sc_gather(embg_p, graph_tok, _SEQ * _N_GT, _DP)

    x1 = _gru2_tc(xt.reshape(_SEQ, _N_TREE, _DP), h0t[0], h0t[1],
                  wih0_t, whh0_t, wih1_t, whh1_t,
                  bih0_t, bhh0_t, bih1_t, bhh1_t, 512)
    hng = _gru2_tc(xg.reshape(_SEQ, _N_GT, _DP), h0g[0], h0g[1],
                   wih0_g, whh0_g, wih1_g, whh1_g,
                   bih0_g, bhh0_g, bih1_g, bhh1_g, 512)

    y1t = _gcn_pre(x1, tc1_Wp, degp_t, 2048)
    tp1 = _sc_agg(y1t, t_src, t_dst, _N_TREE, _DP, _E_TREE)
    y2t = _tree_mid(tp1, y1t, degp_t, tc1_bp, x1[:_BATCH],
                    idx_i[:, None], tc2_Wa, tc2_Wb)
    tp2 = _sc_agg(y2t, t_src, t_dst, _N_TREE, _DP, _E_TREE)
    temb = _tree_post(tp2, y2t, degp_t, tc2_bp, idx_i[None, :])

    ue = _ue_tc(user_feats, ue_W1, ue_b1, ue_W2, ue_b2)
    x_input = jnp.concatenate(
        [temb[:, :_H], ue, hng[_BATCH:, :_H]], axis=0)
    y1g = _gcn_pre(x_input, _pad2(gc1_W, gc1_W.shape[0], _DP), degp_g, 1024)
    gp1 = _sc_agg(y1g, g_src, g_dst, _N_GRAPH, _DP, _E_GRAPH)
    y2g = _gcn_mid_graph(gp1, y1g, degp_g, _pad2(gc1_b[None], 1, _DP), gc2_Wp, 1024)
    gp2 = _sc_agg(y2g, g_src, g_dst, _N_GRAPH, _DP, _E_GRAPH)
    out = _graph_head(gp2, y2g, degp_g, gc2_bp, fc_Wp, fc_b[None])
    return out


# speedup vs baseline: 7.7330x; 1.0001x over previous
import functools

import jax
import jax.numpy as jnp
from jax import lax
from jax.experimental import pallas as pl
from jax.experimental.pallas import tpu as pltpu
from jax.experimental.pallas import tpu_sc as plsc

F32 = jnp.float32

_N_USERS = 2048
_N_GT = 4096
_N_TREE = 2048
_VOCAB = 30000
_D = 100
_H = 100
_SEQ = 20
_E_GRAPH = 65536
_E_TREE = 2048
_BATCH = 32
_N_GRAPH = _N_GT + _N_USERS

_DP = 128
_GP = 128
_G3 = 3 * _GP

_NC = 2
_NS = 16
_NW = _NC * _NS


def _sc_mesh():
    return plsc.VectorSubcoreMesh(core_axis_name="c", subcore_axis_name="s",
                                  num_cores=_NC, num_subcores=_NS)


def _sc_gather(table, idx, B, Dp):
    bpw = B // _NW
    K = min(128, bpw)
    nch = bpw // K

    def body(table_hbm, idx_hbm, out_hbm, idx_v, rows_v, sem):
        c = lax.axis_index("c")
        s = lax.axis_index("s")
        base = (s * _NC + c) * bpw

        def step(j, carry):
            off = base + j * K
            pltpu.sync_copy(idx_hbm.at[pl.ds(off, K)], idx_v)
            pltpu.async_copy(table_hbm.at[idx_v], rows_v, sem).wait()
            pltpu.sync_copy(rows_v, out_hbm.at[pl.ds(off, K)])
            return carry

        lax.fori_loop(0, nch, step, 0)

    k = pl.kernel(
        body,
        out_type=jax.ShapeDtypeStruct((B, Dp), F32),
        mesh=_sc_mesh(),
        compiler_params=pltpu.CompilerParams(use_tc_tiling_on_sc=True),
        scratch_types=[pltpu.VMEM((K,), jnp.int32),
                       pltpu.VMEM((K, Dp), F32),
                       pltpu.SemaphoreType.DMA])
    return k(table, idx)


def _sc_agg(y, src, dst, N, F, E):
    epw = E // _NW
    K = min(128, epw)
    nch = epw // K
    rpt = N // _NS
    zeros = jnp.zeros((N, F), F32)

    def body(y_hbm, src_hbm, dst_hbm, z_hbm, out_hbm,
             si0, si1, di_v, rows0, rows1, acc_sh, sem0, sem1):
        c = lax.axis_index("c")
        s = lax.axis_index("s")
        base = (s * _NC + c) * epw
        zslc = pl.ds(s * rpt, rpt)
        pltpu.sync_copy(z_hbm.at[zslc], acc_sh.at[zslc])
        plsc.subcore_barrier()

        def start(j, si, rows, sem):
            off = base + j * K
            pltpu.sync_copy(src_hbm.at[pl.ds(off, K)], si)
            pltpu.async_copy(y_hbm.at[si], rows, sem)

        def finish(j, rows):
            off = base + j * K
            pltpu.sync_copy(dst_hbm.at[pl.ds(off, K)], di_v)
            pltpu.sync_copy(rows, acc_sh.at[di_v], add=True)

        if nch == 1:
            start(0, si0, rows0, sem0)
            pltpu.make_async_copy(y_hbm.at[si0], rows0, sem0).wait()
            finish(0, rows0)
        else:
            start(0, si0, rows0, sem0)

            def step2(k2, carry):
                j0 = 2 * k2
                pltpu.make_async_copy(y_hbm.at[si0], rows0, sem0).wait()
                start(jnp.minimum(j0 + 1, nch - 1), si1, rows1, sem1)
                finish(j0, rows0)
                pltpu.make_async_copy(y_hbm.at[si1], rows1, sem1).wait()
                start(jnp.minimum(j0 + 2, nch - 1), si0, rows0, sem0)
                finish(j0 + 1, rows1)
                return carry

            lax.fori_loop(0, nch // 2, step2, 0)
            pltpu.make_async_copy(y_hbm.at[si0], rows0, sem0).wait()
        plsc.subcore_barrier()
        pltpu.sync_copy(acc_sh.at[zslc], out_hbm.at[c, zslc])

    k = pl.kernel(
        body,
        out_type=jax.ShapeDtypeStruct((_NC, N, F), F32),
        mesh=_sc_mesh(),
        scratch_types=[pltpu.VMEM((K,), jnp.int32),
                       pltpu.VMEM((K,), jnp.int32),
                       pltpu.VMEM((K,), jnp.int32),
                       pltpu.VMEM((K, F), F32),
                       pltpu.VMEM((K, F), F32),
                       pltpu.VMEM_SHARED((N, F), F32),
                       pltpu.SemaphoreType.DMA,
                       pltpu.SemaphoreType.DMA])
    return k(y, src, dst, zeros)


def _sc_deg(dst, N, E):
    epw = E // _NW
    K = min(128, epw)
    nch = epw // K
    rpt = N // _NS
    ones = jnp.ones((K, _DP), F32)
    zeros = jnp.zeros((N, _DP), F32)

    def body(ones_hbm, z_hbm, dst_hbm, out_hbm, di_v, ones_v, acc_sh):
        c = lax.axis_index("c")
        s = lax.axis_index("s")
        base = (s * _NC + c) * epw
        pltpu.sync_copy(ones_hbm, ones_v)
        zslc = pl.ds(s * rpt, rpt)
        pltpu.sync_copy(z_hbm.at[zslc], acc_sh.at[zslc])
        plsc.subcore_barrier()

        def step(j, carry):
            off = base + j * K
            pltpu.sync_copy(dst_hbm.at[pl.ds(off, K)], di_v)
            pltpu.sync_copy(ones_v, acc_sh.at[di_v], add=True)
            return carry

        lax.fori_loop(0, nch, step, 0)
        plsc.subcore_barrier()
        pltpu.sync_copy(acc_sh.at[zslc], out_hbm.at[c, zslc])

    k = pl.kernel(
        body,
        out_type=jax.ShapeDtypeStruct((_NC, N, _DP), F32),
        mesh=_sc_mesh(),
        scratch_types=[pltpu.VMEM((K,), jnp.int32),
                       pltpu.VMEM((K, _DP), F32),
                       pltpu.VMEM_SHARED((N, _DP), F32)])
    return k(ones, zeros, dst)


def _pad_cols_tc(a, cols_out, bm):
    R, C = a.shape

    def kfn(a_ref, o_ref):
        o_ref[...] = jnp.concatenate(
            [a_ref[...], jnp.zeros((bm, cols_out - C), F32)], axis=1)

    return pl.pallas_call(
        kfn,
        grid=(R // bm,),
        in_specs=[pl.BlockSpec((bm, C), lambda i: (i, 0))],
        out_specs=pl.BlockSpec((bm, cols_out), lambda i: (i, 0)),
        out_shape=jax.ShapeDtypeStruct((R, cols_out), F32))(a)

def _gru2_tc(x, h01, h02, wih0, whh0, wih1, whh1,
             bih0, bhh0, bih1, bhh1, nb):
    S, N, Dx = x.shape

    def kfn(x_ref, h01_ref, h02_ref, wi0_ref, w0_ref, w1_ref, w2_ref,
            bi0_ref, b0_ref, b1_ref, b2_ref, o_ref):
        bf = jnp.bfloat16
        h1 = h01_ref[...]
        h2 = h02_ref[...]
        wi0 = wi0_ref[...].astype(bf)
        w0 = w0_ref[...].astype(bf)
        w1 = w1_ref[...].astype(bf)
        w2 = w2_ref[...].astype(bf)
        bi0 = bi0_ref[...]
        b0 = b0_ref[...]
        b1 = b1_ref[...]
        b2 = b2_ref[...]

        def gate(gi_t, gh_t, h):
            r = jax.nn.sigmoid(gi_t[:, 0:_GP] + gh_t[:, 0:_GP])
            z = jax.nn.sigmoid(gi_t[:, _GP:2 * _GP] + gh_t[:, _GP:2 * _GP])
            n = jnp.tanh(gi_t[:, 2 * _GP:] + r * gh_t[:, 2 * _GP:])
            return (1.0 - z) * n + z * h

        for t in range(S):
            gi1 = jnp.dot(x_ref[t].astype(bf), wi0,
                          preferred_element_type=F32) + bi0
            gh1 = jnp.dot(h1.astype(bf), w0, preferred_element_type=F32) + b0
            h1 = gate(gi1, gh1, h1)
            gi2 = jnp.dot(h1.astype(bf), w1, preferred_element_type=F32) + b1
            gh2 = jnp.dot(h2.astype(bf), w2, preferred_element_type=F32) + b2
            h2 = gate(gi2, gh2, h2)
        o_ref[...] = h2

    wispec = pl.BlockSpec((Dx, _G3), lambda i: (0, 0))
    wspec = pl.BlockSpec((_GP, _G3), lambda i: (0, 0))
    bspec = pl.BlockSpec((1, _G3), lambda i: (0, 0))
    hspec = pl.BlockSpec((nb, _GP), lambda i: (i, 0))
    return pl.pallas_call(
        kfn,
        grid=(N // nb,),
        in_specs=[pl.BlockSpec((S, nb, Dx), lambda i: (0, i, 0)),
                  hspec, hspec, wispec, wspec, wspec, wspec,
                  bspec, bspec, bspec, bspec],
        out_specs=hspec,
        out_shape=jax.ShapeDtypeStruct((N, _GP), F32))(
            x, h01, h02, wih0, whh0, wih1, whh1,
            bih0[None], bhh0[None], bih1[None], bhh1[None])


def _ue_tc(uf, w1, b1, w2, b2):
    def kfn(x_ref, w1_ref, b1_ref, w2_ref, b2_ref, o_ref):
        h = jnp.maximum(jnp.dot(x_ref[...], w1_ref[...],
                                preferred_element_type=F32) + b1_ref[...], 0.0)
        o_ref[...] = jnp.dot(h, w2_ref[...],
                             preferred_element_type=F32) + b2_ref[...]

    return pl.pallas_call(
        kfn,
        out_shape=jax.ShapeDtypeStruct((uf.shape[0], w2.shape[1]), F32))(
            uf, w1, b1[None], w2, b2[None])


def _dinv_of(d):
    return lax.rsqrt(d[0, :, 0:1] + d[1, :, 0:1] + 1.0)


def _elu(x):
    return jnp.where(x > 0, x, jnp.exp(jnp.minimum(x, 0.0)) - 1.0)


def _gcn_pre(x, w, degp, bm):
    B, Kd = x.shape
    Fo = w.shape[1]

    def kfn(x_ref, w_ref, d_ref, o_ref):
        o_ref[...] = _dinv_of(d_ref[...]) * jnp.dot(
            x_ref[...], w_ref[...], preferred_element_type=F32)

    return pl.pallas_call(
        kfn,
        grid=(B // bm,),
        in_specs=[pl.BlockSpec((bm, Kd), lambda i: (i, 0)),
                  pl.BlockSpec((Kd, Fo), lambda i: (0, 0)),
                  pl.BlockSpec((2, bm, _DP), lambda i: (0, i, 0))],
        out_specs=pl.BlockSpec((bm, Fo), lambda i: (i, 0)),
        out_shape=jax.ShapeDtypeStruct((B, Fo), F32))(x, w, degp)


def _gcn_mid_graph(p, y1, degp, b1, w2, bm):
    _, B, F1 = p.shape
    F2 = w2.shape[1]

    def kfn(p_ref, y_ref, d_ref, b_ref, w_ref, o_ref):
        dinv = _dinv_of(d_ref[...])
        xg = _elu(dinv * (p_ref[0] + p_ref[1] + y_ref[...]) + b_ref[...])
        o_ref[...] = dinv * jnp.dot(xg, w_ref[...], preferred_element_type=F32)

    return pl.pallas_call(
        kfn,
        grid=(B // bm,),
        in_specs=[pl.BlockSpec((2, bm, F1), lambda i: (0, i, 0)),
                  pl.BlockSpec((bm, F1), lambda i: (i, 0)),
                  pl.BlockSpec((2, bm, _DP), lambda i: (0, i, 0)),
                  pl.BlockSpec((1, F1), lambda i: (0, 0)),
                  pl.BlockSpec((F1, F2), lambda i: (0, 0))],
        out_specs=pl.BlockSpec((bm, F2), lambda i: (i, 0)),
        out_shape=jax.ShapeDtypeStruct((B, F2), F32))(p, y1, degp, b1, w2)


def _graph_head(p, y2, degp, b2, fcw, fcb):
    _, _, F2 = p.shape
    C = fcw.shape[1]

    def kfn(p_ref, y_ref, d_ref, b_ref, w_ref, fb_ref, o_ref):
        dinv = _dinv_of(d_ref[...])
        xg = _elu(dinv * (p_ref[0] + p_ref[1] + y_ref[...]) + b_ref[...])
        o_ref[...] = jnp.dot(xg, w_ref[...],
                             preferred_element_type=F32) + fb_ref[...]

    return pl.pallas_call(
        kfn,
        grid=(1,),
        in_specs=[pl.BlockSpec((2, _BATCH, F2), lambda i: (0, 0, 0)),
                  pl.BlockSpec((_BATCH, F2), lambda i: (0, 0)),
                  pl.BlockSpec((2, _BATCH, _DP), lambda i: (0, 0, 0)),
                  pl.BlockSpec((1, F2), lambda i: (0, 0)),
                  pl.BlockSpec((F2, C), lambda i: (0, 0)),
                  pl.BlockSpec((1, C), lambda i: (0, 0))],
        out_specs=pl.BlockSpec((_BATCH, C), lambda i: (0, 0)),
        out_shape=jax.ShapeDtypeStruct((_BATCH, C), F32))(
            p, y2, degp, b2, fcw, fcb)


def _tree_mid(p, y1, degp, b1, x1head, idxcol, wa, wb):
    N = y1.shape[0]

    def kfn(p_ref, y_ref, d_ref, b_ref, xh_ref, idx_ref, wa_ref, wb_ref, o_ref):
        dinv = _dinv_of(d_ref[...])
        xca = _elu(dinv * (p_ref[0] + p_ref[1] + y_ref[...]) + b_ref[...])
        cols = lax.broadcasted_iota(jnp.int32, (N, _BATCH), 1)
        oh = (cols == idx_ref[...]).astype(F32)
        xcb = _elu(jnp.dot(oh, xh_ref[...], preferred_element_type=F32))
        o_ref[...] = dinv * (
            jnp.dot(xca, wa_ref[...], preferred_element_type=F32)
            + jnp.dot(xcb, wb_ref[...], preferred_element_type=F32))

    return pl.pallas_call(
        kfn,
        out_shape=jax.ShapeDtypeStruct((N, _DP), F32))(
            p, y1, degp, b1, x1head, idxcol, wa, wb)


def _tree_post(p, y2, degp, b2, idxrow):
    N = y2.shape[0]

    def kfn(p_ref, y_ref, d_ref, b_ref, idx_ref, o_ref):
        dinv = _dinv_of(d_ref[...])
        xc2 = _elu(dinv * (p_ref[0] + p_ref[1] + y_ref[...]) + b_ref[...])
        rows = lax.broadcasted_iota(jnp.int32, (_BATCH, N), 0)
        oht = (rows == idx_ref[...]).astype(F32)
        seg = jnp.dot(oht, xc2, preferred_element_type=F32)
        cnt = jnp.sum(oht, axis=1, keepdims=True)
        o_ref[...] = seg / cnt

    return pl.pallas_call(
        kfn,
        out_shape=jax.ShapeDtypeStruct((_BATCH, _DP), F32))(
            p, y2, degp, b2, idxrow)


def _pad2(a, r, c):
    return jnp.pad(a, ((0, r - a.shape[0]), (0, c - a.shape[1])))


def _gates_T(W, kpad):
    wt = W.T.reshape(W.shape[1], 3, _H)
    wt = jnp.pad(wt, ((0, kpad - W.shape[1]), (0, 0), (0, _GP - _H)))
    return wt.reshape(kpad, _G3)


def _gates_b(b):
    return jnp.pad(b.reshape(3, _H), ((0, 0), (0, _GP - _H))).reshape(_G3)


def kernel(user_feats, graph_node_features, graph_edge_index,
           merged_tree_feature, merged_tree_edge_index, indices,
           emb_tree, emb_graph, h0_tree, h0_graph,
           ue_W1, ue_b1, ue_W2, ue_b2,
           gt_Wih0, gt_Whh0, gt_bih0, gt_bhh0,
           gt_Wih1, gt_Whh1, gt_bih1, gt_bhh1,
           gg_Wih0, gg_Whh0, gg_bih0, gg_bhh0,
           gg_Wih1, gg_Whh1, gg_bih1, gg_bhh1,
           tc1_W, tc1_b, tc2_W, tc2_b,
           gc1_W, gc1_b, gc2_W, gc2_b,
           fc_W, fc_b):
    i32 = jnp.int32
    tree_tok = merged_tree_feature.astype(i32).T.reshape(-1)
    graph_tok = graph_node_features.astype(i32).T.reshape(-1)
    embt_p = _pad_cols_tc(emb_tree, _DP, 3000)
    embg_p = _pad_cols_tc(emb_graph, _DP, 3000)
    g_src = graph_edge_index[0].astype(i32)
    g_dst = graph_edge_index[1].astype(i32)
    t_src = merged_tree_edge_index[1].astype(i32)
    t_dst = merged_tree_edge_index[0].astype(i32)
    idx_i = indices.astype(i32)
    h0t = jnp.pad(h0_tree, ((0, 0), (0, 0), (0, _GP - _H)))
    h0g = jnp.pad(h0_graph, ((0, 0), (0, 0), (0, _GP - _H)))

    wih0_t = _gates_T(gt_Wih0, _DP)
    whh0_t = _gates_T(gt_Whh0, _GP)
    wih1_t = _gates_T(gt_Wih1, _GP)
    whh1_t = _gates_T(gt_Whh1, _GP)
    wih0_g = _gates_T(gg_Wih0, _DP)
    whh0_g = _gates_T(gg_Whh0, _GP)
    wih1_g = _gates_T(gg_Wih1, _GP)
    whh1_g = _gates_T(gg_Whh1, _GP)
    bih0_t = _gates_b(gt_bih0)
    bhh0_t = _gates_b(gt_bhh0)
    bih1_t = _gates_b(gt_bih1)
    bhh1_t = _gates_b(gt_bhh1)
    bih0_g = _gates_b(gg_bih0)
    bhh0_g = _gates_b(gg_bhh0)
    bih1_g = _gates_b(gg_bih1)
    bhh1_g = _gates_b(gg_bhh1)

    tc1_Wp = _pad2(tc1_W, _GP, _DP)
    tc2_Wa = _pad2(tc2_W[:_H], _DP, _DP)
    tc2_Wb = _pad2(tc2_W[_H:], _GP, _DP)
    tc1_bp = _pad2(tc1_b[None], 1, _DP)
    tc2_bp = _pad2(tc2_b[None], 1, _DP)
    gc2_Wp = _pad2(gc2_W, _DP, _DP)
    gc2_bp = _pad2(gc2_b[None], 1, _DP)
    fc_Wp = _pad2(fc_W, _DP, fc_W.shape[1])

    degp_g = _sc_deg(g_dst, _N_GRAPH, _E_GRAPH)
    degp_t = _sc_deg(t_dst, _N_TREE, _E_TREE)
    xt = _sc_gather(embt_p, tree_tok, _SEQ * _N_TREE, _DP)
    xg = _sc_gather(embg_p, graph_tok, _SEQ * _N_GT, _DP)

    x1 = _gru2_tc(xt.reshape(_SEQ, _N_TREE, _DP), h0t[0], h0t[1],
                  wih0_t, whh0_t, wih1_t, whh1_t,
                  bih0_t, bhh0_t, bih1_t, bhh1_t, 512)
    hng = _gru2_tc(xg.reshape(_SEQ, _N_GT, _DP), h0g[0], h0g[1],
                   wih0_g, whh0_g, wih1_g, whh1_g,
                   bih0_g, bhh0_g, bih1_g, bhh1_g, 512)

    y1t = _gcn_pre(x1, tc1_Wp, degp_t, 2048)
    tp1 = _sc_agg(y1t, t_src, t_dst, _N_TREE, _DP, _E_TREE)
    y2t = _tree_mid(tp1, y1t, degp_t, tc1_bp, x1[:_BATCH],
                    idx_i[:, None], tc2_Wa, tc2_Wb)
    tp2 = _sc_agg(y2t, t_src, t_dst, _N_TREE, _DP, _E_TREE)
    temb = _tree_post(tp2, y2t, degp_t, tc2_bp, idx_i[None, :])

    ue = _ue_tc(user_feats, ue_W1, ue_b1, ue_W2, ue_b2)
    x_input = jnp.concatenate(
        [temb[:, :_H], ue, hng[_BATCH:, :_H]], axis=0)
    y1g = _gcn_pre(x_input, _pad2(gc1_W, gc1_W.shape[0], _DP), degp_g, 1024)
    gp1 = _sc_agg(y1g, g_src, g_dst, _N_GRAPH, _DP, _E_GRAPH)
    y2g = _gcn_mid_graph(gp1, y1g, degp_g, _pad2(gc1_b[None], 1, _DP), gc2_Wp, 1024)
    gp2 = _sc_agg(y2g, g_src, g_dst, _N_GRAPH, _DP, _E_GRAPH)
    out = _graph_head(gp2, y2g, degp_g, gc2_bp, fc_Wp, fc_b[None])
    return out
